# Initial kernel scaffold; baseline (speedup 1.0000x reference)
import jax
import jax.numpy as jnp
from jax.experimental import pallas as pl


def kernel(x, edge_index0, edge_index1, W_in, b_in, fc_W0, bias0, attn_l0, attn_r0, res_W0, ln_g, ln_b, fc_W1, bias1, attn_l1, attn_r1):
    raise NotImplementedError("write your pallas kernel here")


# speedup vs baseline: 4.0649x; 4.0649x over previous
import jax
import jax.numpy as jnp
from jax.experimental import pallas as pl
from jax.experimental.pallas import tpu as pltpu

N = 10000
E = 320000
D_IN = 128
HID = 64
HEADS = 4
OUT = 64
NEG_SLOPE = 0.2

_BLK = 400


def _proj0_body(x_ref, Win_ref, bin_ref, fcW_ref, resW_ref, al_ref, ar_ref,
                h_ref, feat_ref, res_ref, el_ref, er_ref):
    x = x_ref[...]
    h = jnp.dot(x, Win_ref[...], preferred_element_type=jnp.float32) + bin_ref[...]
    h_ref[...] = h
    feat = jnp.dot(h, fcW_ref[...], preferred_element_type=jnp.float32)
    feat_ref[...] = feat
    res_ref[...] = jnp.dot(h, resW_ref[...], preferred_element_type=jnp.float32)
    f = feat.reshape(_BLK, HEADS, HID)
    el_ref[...] = jnp.sum(f * al_ref[...][None], axis=-1)
    er_ref[...] = jnp.sum(f * ar_ref[...][None], axis=-1)


def _proj0(x, W_in, b_in, fc_W0, res_W0, al0, ar0):
    grid = N // _BLK
    return pl.pallas_call(
        _proj0_body,
        grid=(grid,),
        in_specs=[
            pl.BlockSpec((_BLK, D_IN), lambda i: (i, 0)),
            pl.BlockSpec((D_IN, HID), lambda i: (0, 0)),
            pl.BlockSpec((HID,), lambda i: (0,)),
            pl.BlockSpec((HID, HEADS * HID), lambda i: (0, 0)),
            pl.BlockSpec((HID, HEADS * HID), lambda i: (0, 0)),
            pl.BlockSpec((HEADS, HID), lambda i: (0, 0)),
            pl.BlockSpec((HEADS, HID), lambda i: (0, 0)),
        ],
        out_specs=[
            pl.BlockSpec((_BLK, HID), lambda i: (i, 0)),
            pl.BlockSpec((_BLK, HEADS * HID), lambda i: (i, 0)),
            pl.BlockSpec((_BLK, HEADS * HID), lambda i: (i, 0)),
            pl.BlockSpec((_BLK, HEADS), lambda i: (i, 0)),
            pl.BlockSpec((_BLK, HEADS), lambda i: (i, 0)),
        ],
        out_shape=[
            jax.ShapeDtypeStruct((N, HID), jnp.float32),
            jax.ShapeDtypeStruct((N, HEADS * HID), jnp.float32),
            jax.ShapeDtypeStruct((N, HEADS * HID), jnp.float32),
            jax.ShapeDtypeStruct((N, HEADS), jnp.float32),
            jax.ShapeDtypeStruct((N, HEADS), jnp.float32),
        ],
    )(x.astype(jnp.float32), W_in, b_in, fc_W0, res_W0, al0, ar0)


def _mid_body(S_ref, den_ref, res_ref, bias_ref, lng_ref, lnb_ref,
              fcW1_ref, al1_ref, ar1_ref,
              feat1_ref, el1_ref, er1_ref):
    S = S_ref[...]
    den = jnp.maximum(den_ref[...], 1e-9)
    den = jnp.repeat(den, HID, axis=1)
    rst = S / den + res_ref[...] + bias_ref[...]
    mu = jnp.mean(rst, axis=-1, keepdims=True)
    var = jnp.mean((rst - mu) ** 2, axis=-1, keepdims=True)
    hn = (rst - mu) / jnp.sqrt(var + 1e-5) * lng_ref[...] + lnb_ref[...]
    h = jnp.where(hn > 0, hn, jnp.exp(jnp.minimum(hn, 0.0)) - 1.0)
    feat = jnp.dot(h, fcW1_ref[...], preferred_element_type=jnp.float32)
    feat1_ref[...] = feat
    el1_ref[...] = jnp.sum(feat * al1_ref[...], axis=-1, keepdims=True)
    er1_ref[...] = jnp.sum(feat * ar1_ref[...], axis=-1, keepdims=True)


def _mid(S0, den0, res0, bias0, ln_g, ln_b, fc_W1, al1, ar1):
    grid = N // _BLK
    return pl.pallas_call(
        _mid_body,
        grid=(grid,),
        in_specs=[
            pl.BlockSpec((_BLK, HEADS * HID), lambda i: (i, 0)),
            pl.BlockSpec((_BLK, HEADS), lambda i: (i, 0)),
            pl.BlockSpec((_BLK, HEADS * HID), lambda i: (i, 0)),
            pl.BlockSpec((1, HEADS * HID), lambda i: (0, 0)),
            pl.BlockSpec((1, HEADS * HID), lambda i: (0, 0)),
            pl.BlockSpec((1, HEADS * HID), lambda i: (0, 0)),
            pl.BlockSpec((HEADS * HID, OUT), lambda i: (0, 0)),
            pl.BlockSpec((1, OUT), lambda i: (0, 0)),
            pl.BlockSpec((1, OUT), lambda i: (0, 0)),
        ],
        out_specs=[
            pl.BlockSpec((_BLK, OUT), lambda i: (i, 0)),
            pl.BlockSpec((_BLK, 1), lambda i: (i, 0)),
            pl.BlockSpec((_BLK, 1), lambda i: (i, 0)),
        ],
        out_shape=[
            jax.ShapeDtypeStruct((N, OUT), jnp.float32),
            jax.ShapeDtypeStruct((N, 1), jnp.float32),
            jax.ShapeDtypeStruct((N, 1), jnp.float32),
        ],
    )(S0, den0, res0, bias0, ln_g, ln_b, fc_W1, al1, ar1)


def _fin_body(S_ref, den_ref, bias_ref, out_ref):
    den = jnp.maximum(den_ref[...], 1e-9)
    out_ref[...] = S_ref[...] / den + bias_ref[...]


def _fin(S1, den1, bias1):
    grid = N // _BLK
    return pl.pallas_call(
        _fin_body,
        grid=(grid,),
        in_specs=[
            pl.BlockSpec((_BLK, OUT), lambda i: (i, 0)),
            pl.BlockSpec((_BLK, 1), lambda i: (i, 0)),
            pl.BlockSpec((1, OUT), lambda i: (0, 0)),
        ],
        out_specs=pl.BlockSpec((_BLK, OUT), lambda i: (i, 0)),
        out_shape=jax.ShapeDtypeStruct((N, OUT), jnp.float32),
    )(S1, den1, bias1)


def _edge_xla(feat, el, er, src, dst, heads, dim):
    gmax = jnp.max(el, axis=0)
    C = jnp.where(gmax[None] + er > 0, gmax[None] + er,
                  NEG_SLOPE * (gmax[None] + er))
    s = el[src] + er[dst]
    e = jnp.where(s > 0, s, NEG_SLOPE * s)
    ee = jnp.exp(e - C[dst])
    den = jax.ops.segment_sum(ee, dst, num_segments=N)
    msg = feat[src].reshape(E, heads, dim) * ee[:, :, None]
    S = jax.ops.segment_sum(msg.reshape(E, heads * dim), dst, num_segments=N)
    return S, den


def kernel(x, edge_index0, edge_index1, W_in, b_in, fc_W0, bias0, attn_l0,
           attn_r0, res_W0, ln_g, ln_b, fc_W1, bias1, attn_l1, attn_r1):
    src0, dst0 = edge_index0[0], edge_index0[1]
    src1, dst1 = edge_index1[0], edge_index1[1]

    h, feat0, res0, el0, er0 = _proj0(x, W_in, b_in, fc_W0, res_W0,
                                      attn_l0, attn_r0)
    S0, den0 = _edge_xla(feat0, el0, er0, src0, dst0, HEADS, HID)
    feat1, el1, er1 = _mid(S0, den0, res0, bias0.reshape(1, HEADS * HID),
                           ln_g.reshape(1, -1), ln_b.reshape(1, -1),
                           fc_W1, attn_l1, attn_r1)
    S1, den1 = _edge_xla(feat1, el1, er1, src1, dst1, 1, OUT)
    out = _fin(S1, den1, bias1.reshape(1, OUT))
    return out


# speedup vs baseline: 25.3348x; 6.2327x over previous
import functools

import jax
import jax.numpy as jnp
from jax import lax
from jax.experimental import pallas as pl
from jax.experimental.pallas import tpu as pltpu
import jax.experimental.pallas.tpu_sc as plsc

N = 10000
NP = 10240
E = 320000
D_IN = 128
HID = 64
HEADS = 4
OUT = 64
NEG_SLOPE = 0.2

_BLK = 400
_R0 = 144
_R1 = 80
_NB = E // 128
_RPT = NP // 16


def _lrelu(x):
    return jnp.where(x > 0, x, NEG_SLOPE * x)


def _proj0_body(x_ref, Win_ref, bin_ref, fcW_ref, resW_ref, al_ref, ar_ref,
                F_ref, res_ref, elp_ref, erp_ref, gm_ref):
    i = pl.program_id(0)
    x = x_ref[...]
    h = jnp.dot(x, Win_ref[...], preferred_element_type=jnp.float32) + bin_ref[...]
    feat = jnp.dot(h, fcW_ref[...], preferred_element_type=jnp.float32)
    res_ref[...] = jnp.dot(h, resW_ref[...], preferred_element_type=jnp.float32)
    f = feat.reshape(_BLK, HEADS, HID)
    el = jnp.sum(f * al_ref[...][None], axis=-1)
    er = jnp.sum(f * ar_ref[...][None], axis=-1)
    zpad = jnp.zeros((_BLK, 16 - HEADS), jnp.float32)
    elp_ref[...] = jnp.concatenate([el, zpad], axis=1)
    erp_ref[...] = jnp.concatenate([er, zpad], axis=1)
    @pl.when(i == 0)
    def _():
        gm_ref[...] = jnp.full((1, 16), -1e30, jnp.float32)
    gm_ref[...] = jnp.maximum(gm_ref[...], jnp.max(el))
    ones = jnp.ones((_BLK, 2), jnp.float32)
    fpad = jnp.zeros((_BLK, _R0 - 2 * HID - 2), jnp.float32)
    F_ref[0] = jnp.concatenate([feat[:, :2 * HID], ones, fpad], axis=1)
    F_ref[1] = jnp.concatenate([feat[:, 2 * HID:], ones, fpad], axis=1)


def _proj0(x, W_in, b_in, fc_W0, res_W0, al0, ar0):
    return pl.pallas_call(
        _proj0_body,
        grid=(N // _BLK,),
        in_specs=[
            pl.BlockSpec((_BLK, D_IN), lambda i: (i, 0)),
            pl.BlockSpec((D_IN, HID), lambda i: (0, 0)),
            pl.BlockSpec((HID,), lambda i: (0,)),
            pl.BlockSpec((HID, HEADS * HID), lambda i: (0, 0)),
            pl.BlockSpec((HID, HEADS * HID), lambda i: (0, 0)),
            pl.BlockSpec((HEADS, HID), lambda i: (0, 0)),
            pl.BlockSpec((HEADS, HID), lambda i: (0, 0)),
        ],
        out_specs=[
            pl.BlockSpec((2, _BLK, _R0), lambda i: (0, i, 0)),
            pl.BlockSpec((_BLK, HEADS * HID), lambda i: (i, 0)),
            pl.BlockSpec((_BLK, 16), lambda i: (i, 0)),
            pl.BlockSpec((_BLK, 16), lambda i: (i, 0)),
            pl.BlockSpec((1, 16), lambda i: (0, 0)),
        ],
        out_shape=[
            jax.ShapeDtypeStruct((2, N, _R0), jnp.float32),
            jax.ShapeDtypeStruct((N, HEADS * HID), jnp.float32),
            jax.ShapeDtypeStruct((N, 16), jnp.float32),
            jax.ShapeDtypeStruct((N, 16), jnp.float32),
            jax.ShapeDtypeStruct((1, 16), jnp.float32),
        ],
    )(x, W_in, b_in, fc_W0, res_W0, al0, ar0)


def _mid_body(Sp_ref, res_ref, bias_ref, lng_ref, lnb_ref,
              fcW1_ref, al1_ref, ar1_ref,
              F_ref, elp_ref, erp_ref, gm_ref):
    i = pl.program_id(0)
    Sa = Sp_ref[0]
    Sb = Sp_ref[1]
    den = jnp.concatenate([Sa[:, 2 * HID:2 * HID + 2],
                           Sb[:, 2 * HID:2 * HID + 2]], axis=1)
    inv = 1.0 / jnp.maximum(den, 1e-9)
    inv = jnp.repeat(inv, HID, axis=1)
    S = jnp.concatenate([Sa[:, :2 * HID], Sb[:, :2 * HID]], axis=1)
    rst = S * inv + res_ref[...] + bias_ref[...]
    mu = jnp.mean(rst, axis=-1, keepdims=True)
    var = jnp.mean((rst - mu) ** 2, axis=-1, keepdims=True)
    hn = (rst - mu) / jnp.sqrt(var + 1e-5) * lng_ref[...] + lnb_ref[...]
    h = jnp.where(hn > 0, hn, jnp.exp(jnp.minimum(hn, 0.0)) - 1.0)
    feat = jnp.dot(h, fcW1_ref[...], preferred_element_type=jnp.float32)
    el = jnp.sum(feat * al1_ref[...], axis=-1, keepdims=True)
    er = jnp.sum(feat * ar1_ref[...], axis=-1, keepdims=True)
    zpad = jnp.zeros((_BLK, 15), jnp.float32)
    elp_ref[...] = jnp.concatenate([el, zpad], axis=1)
    erp_ref[...] = jnp.concatenate([er, zpad], axis=1)
    @pl.when(i == 0)
    def _():
        gm_ref[...] = jnp.full((1, 16), -1e30, jnp.float32)
    gm_ref[...] = jnp.maximum(gm_ref[...], jnp.max(el))
    ones = jnp.ones((_BLK, 1), jnp.float32)
    fpad = jnp.zeros((_BLK, _R1 - OUT - 1), jnp.float32)
    F_ref[...] = jnp.concatenate([feat, ones, fpad], axis=1)


def _mid(S0p, res0, bias0, ln_g, ln_b, fc_W1, al1, ar1):
    return pl.pallas_call(
        _mid_body,
        grid=(N // _BLK,),
        in_specs=[
            pl.BlockSpec((2, _BLK, _R0), lambda i: (0, i, 0)),
            pl.BlockSpec((_BLK, HEADS * HID), lambda i: (i, 0)),
            pl.BlockSpec((1, HEADS * HID), lambda i: (0, 0)),
            pl.BlockSpec((1, HEADS * HID), lambda i: (0, 0)),
            pl.BlockSpec((1, HEADS * HID), lambda i: (0, 0)),
            pl.BlockSpec((HEADS * HID, OUT), lambda i: (0, 0)),
            pl.BlockSpec((1, OUT), lambda i: (0, 0)),
            pl.BlockSpec((1, OUT), lambda i: (0, 0)),
        ],
        out_specs=[
            pl.BlockSpec((_BLK, _R1), lambda i: (i, 0)),
            pl.BlockSpec((_BLK, 16), lambda i: (i, 0)),
            pl.BlockSpec((_BLK, 16), lambda i: (i, 0)),
            pl.BlockSpec((1, 16), lambda i: (0, 0)),
        ],
        out_shape=[
            jax.ShapeDtypeStruct((N, _R1), jnp.float32),
            jax.ShapeDtypeStruct((N, 16), jnp.float32),
            jax.ShapeDtypeStruct((N, 16), jnp.float32),
            jax.ShapeDtypeStruct((1, 16), jnp.float32),
        ],
    )(S0p, res0, bias0, ln_g, ln_b, fc_W1, al1, ar1)


def _fin_body(Sp_ref, bias_ref, out_ref):
    agg = Sp_ref[0] + Sp_ref[1]
    den = jnp.maximum(agg[:, OUT:OUT + 1], 1e-9)
    out_ref[...] = agg[:, :OUT] / den + bias_ref[...]


def _fin(S1p, bias1):
    return pl.pallas_call(
        _fin_body,
        grid=(N // _BLK,),
        in_specs=[
            pl.BlockSpec((2, _BLK, _R1), lambda i: (0, i, 0)),
            pl.BlockSpec((1, OUT), lambda i: (0, 0)),
        ],
        out_specs=pl.BlockSpec((_BLK, OUT), lambda i: (i, 0)),
        out_shape=jax.ShapeDtypeStruct((N, OUT), jnp.float32),
    )(S1p, bias1)


def _sc_mesh():
    return plsc.VectorSubcoreMesh(core_axis_name="c", subcore_axis_name="s")


_SC_PARAMS = pltpu.CompilerParams(use_tc_tiling_on_sc=False)


def _attn_kernel():
    nb_per_tile = (_NB + 31) // 32

    def body(src_hbm, dst_hbm, elp_hbm, erp_hbm, gm_hbm, ee_hbm,
             src_v, dst_v, el_v, er_v, out_v, gm_v, sem):
        c = lax.axis_index("c")
        s = lax.axis_index("s")
        wid = s * 2 + c
        pltpu.sync_copy(gm_hbm, gm_v)
        gs = gm_v[0, :]

        def batch(ii, _):
            b = ii * 32 + wid

            @pl.when(b < _NB)
            def _():
                pltpu.sync_copy(src_hbm.at[pl.ds(b * 128, 128)], src_v)
                pltpu.sync_copy(dst_hbm.at[pl.ds(b * 128, 128)], dst_v)
                pltpu.async_copy(elp_hbm.at[src_v], el_v, sem).wait()
                pltpu.async_copy(erp_hbm.at[dst_v], er_v, sem).wait()

                def edge(j, _):
                    el = el_v[j, :]
                    er = er_v[j, :]
                    e = _lrelu(el + er)
                    cc = _lrelu(gs + er)
                    out_v[j, :] = jnp.exp(e - cc)
                    return 0

                lax.fori_loop(0, 128, edge, 0)
                pltpu.sync_copy(out_v, ee_hbm.at[pl.ds(b * 128, 128), :])
            return 0

        lax.fori_loop(0, nb_per_tile, batch, 0)

    kern = functools.partial(
        pl.kernel, mesh=_sc_mesh(),
        out_type=jax.ShapeDtypeStruct((E, 16), jnp.float32),
        compiler_params=_SC_PARAMS,
        scratch_types=[
            pltpu.VMEM((128,), jnp.int32),
            pltpu.VMEM((128,), jnp.int32),
            pltpu.VMEM((128, 16), jnp.float32),
            pltpu.VMEM((128, 16), jnp.float32),
            pltpu.VMEM((128, 16), jnp.float32),
            pltpu.VMEM((1, 16), jnp.float32),
            pltpu.SemaphoreType.DMA,
        ],
    )
    return kern(body)


def _agg_kernel(r_width, heads_split):
    nv = r_width // 16
    half_nb = _NB // 2
    n_iter = (_NB + 15) // 16 if heads_split else (half_nb + 15) // 16

    def body(src_hbm, dst_hbm, ee_hbm, F_hbm, out_hbm,
             rows_v, src_v, dst_v, idx_v, ee_v, S_sh, sem):
        c = lax.axis_index("c")
        s = lax.axis_index("s")

        def zrow(j, _):
            for v in range(nv):
                rows_v[j, pl.ds(v * 16, 16)] = jnp.zeros((16,), jnp.float32)
            return 0

        lax.fori_loop(0, 128, zrow, 0)
        for q in range(_RPT // 128):
            pltpu.sync_copy(rows_v,
                            S_sh.at[pl.ds(s * _RPT + q * 128, 128), :])
        plsc.subcore_barrier()

        def batch(ii, _):
            local = ii * 16 + s
            bound = _NB if heads_split else half_nb

            @pl.when(local < bound)
            def _():
                b = local if heads_split else c * half_nb + local
                pltpu.sync_copy(src_hbm.at[pl.ds(b * 128, 128)], src_v)
                pltpu.sync_copy(dst_hbm.at[pl.ds(b * 128, 128)], dst_v)
                pltpu.sync_copy(ee_hbm.at[pl.ds(b * 128, 128), :], ee_v)
                if heads_split:
                    for g in range(8):
                        sv = src_v[pl.ds(g * 16, 16)]
                        idx_v[pl.ds(g * 16, 16)] = sv + c * N
                    gather_ref = F_hbm.at[idx_v]
                else:
                    gather_ref = F_hbm.at[src_v]
                pltpu.async_copy(gather_ref, rows_v, sem).wait()

                def edge(j, _):
                    v = ee_v[j, :]
                    if heads_split:
                        m0 = jnp.where(c == 0, v[0], v[2])
                        m1 = jnp.where(c == 0, v[1], v[3])
                    else:
                        m0 = v[0]
                        m1 = v[0]
                    m0v = jnp.full((16,), m0, jnp.float32)
                    m1v = jnp.full((16,), m1, jnp.float32)
                    half = (nv - 1) // 2 if heads_split else nv - 1
                    for w in range(nv - 1):
                        r = rows_v[j, pl.ds(w * 16, 16)]
                        rows_v[j, pl.ds(w * 16, 16)] = r * (m0v if w < half
                                                            else m1v)
                    io = lax.broadcasted_iota(jnp.int32, (16,), 0)
                    mult = jnp.where(io == 0, m0v,
                                     jnp.where(io == 1, m1v, 0.0))
                    r = rows_v[j, pl.ds((nv - 1) * 16, 16)]
                    rows_v[j, pl.ds((nv - 1) * 16, 16)] = r * mult
                    return 0

                lax.fori_loop(0, 128, edge, 0)
                pltpu.sync_copy(rows_v, S_sh.at[dst_v], add=True)
            return 0

        lax.fori_loop(0, n_iter, batch, 0)
        plsc.subcore_barrier()
        pltpu.sync_copy(S_sh.at[pl.ds(s * _RPT, _RPT), :],
                        out_hbm.at[c, pl.ds(s * _RPT, _RPT), :])

    kern = functools.partial(
        pl.kernel, mesh=_sc_mesh(),
        out_type=jax.ShapeDtypeStruct((2, NP, r_width), jnp.float32),
        compiler_params=_SC_PARAMS,
        scratch_types=[
            pltpu.VMEM((128, r_width), jnp.float32),
            pltpu.VMEM((128,), jnp.int32),
            pltpu.VMEM((128,), jnp.int32),
            pltpu.VMEM((128,), jnp.int32),
            pltpu.VMEM((128, 16), jnp.float32),
            pltpu.VMEM_SHARED((NP, r_width), jnp.float32),
            pltpu.SemaphoreType.DMA,
        ],
    )
    return kern(body)


def kernel(x, edge_index0, edge_index1, W_in, b_in, fc_W0, bias0, attn_l0,
           attn_r0, res_W0, ln_g, ln_b, fc_W1, bias1, attn_l1, attn_r1):
    src0, dst0 = edge_index0[0], edge_index0[1]
    src1, dst1 = edge_index1[0], edge_index1[1]

    F0, res0, elp0, erp0, gm0 = _proj0(x, W_in, b_in, fc_W0, res_W0,
                                       attn_l0, attn_r0)
    ee0 = _attn_kernel()(src0, dst0, elp0, erp0, gm0)
    S0 = _agg_kernel(_R0, True)(src0, dst0, ee0, F0.reshape(2 * N, _R0))
    F1, elp1, erp1, gm1 = _mid(S0, res0,
                               bias0.reshape(1, HEADS * HID),
                               ln_g.reshape(1, -1), ln_b.reshape(1, -1),
                               fc_W1, attn_l1, attn_r1)
    ee1 = _attn_kernel()(src1, dst1, elp1, erp1, gm1)
    S1 = _agg_kernel(_R1, False)(src1, dst1, ee1, F1)
    out = _fin(S1, bias1.reshape(1, OUT))
    return out


# speedup vs baseline: 77.4992x; 3.0590x over previous
import functools

import jax
import jax.numpy as jnp
from jax import lax
from jax.experimental import pallas as pl
from jax.experimental.pallas import tpu as pltpu
import jax.experimental.pallas.tpu_sc as plsc

N = 10000
NP = 10240
E = 320000
D_IN = 128
HID = 64
HEADS = 4
OUT = 64
NEG_SLOPE = 0.2

_BLK = 400
_R0 = 144
_R1 = 80
_NB = E // 128
_EPAD = 2560
_RPT = NP // 16


def _lrelu(x):
    return jnp.where(x > 0, x, NEG_SLOPE * x)


def _proj0_body(x_ref, Win_ref, bin_ref, fcW_ref, resW_ref, al_ref, ar_ref,
                F_ref, res_ref, elp_ref, erp_ref, gm_ref):
    i = pl.program_id(0)
    x = x_ref[...]
    h = jnp.dot(x, Win_ref[...], preferred_element_type=jnp.float32) + bin_ref[...]
    feat = jnp.dot(h, fcW_ref[...], preferred_element_type=jnp.float32)
    res_ref[...] = jnp.dot(h, resW_ref[...], preferred_element_type=jnp.float32)
    f = feat.reshape(_BLK, HEADS, HID)
    el = jnp.sum(f * al_ref[...][None], axis=-1)
    er = jnp.sum(f * ar_ref[...][None], axis=-1)
    zpad = jnp.zeros((_BLK, 16 - HEADS), jnp.float32)
    elp_ref[...] = jnp.concatenate([el, zpad], axis=1)
    erp_ref[...] = jnp.concatenate([er, zpad], axis=1)
    @pl.when(i == 0)
    def _():
        gm_ref[...] = jnp.full((1, 16), -1e30, jnp.float32)
    gm_ref[...] = jnp.maximum(gm_ref[...], jnp.max(el))
    ones = jnp.ones((_BLK, 2), jnp.float32)
    fpad = jnp.zeros((_BLK, _R0 - 2 * HID - 2), jnp.float32)
    F_ref[0] = jnp.concatenate([feat[:, :2 * HID], ones, fpad], axis=1)
    F_ref[1] = jnp.concatenate([feat[:, 2 * HID:], ones, fpad], axis=1)


def _proj0(x, W_in, b_in, fc_W0, res_W0, al0, ar0):
    return pl.pallas_call(
        _proj0_body,
        grid=(N // _BLK,),
        in_specs=[
            pl.BlockSpec((_BLK, D_IN), lambda i: (i, 0)),
            pl.BlockSpec((D_IN, HID), lambda i: (0, 0)),
            pl.BlockSpec((HID,), lambda i: (0,)),
            pl.BlockSpec((HID, HEADS * HID), lambda i: (0, 0)),
            pl.BlockSpec((HID, HEADS * HID), lambda i: (0, 0)),
            pl.BlockSpec((HEADS, HID), lambda i: (0, 0)),
            pl.BlockSpec((HEADS, HID), lambda i: (0, 0)),
        ],
        out_specs=[
            pl.BlockSpec((2, _BLK, _R0), lambda i: (0, i, 0)),
            pl.BlockSpec((_BLK, HEADS * HID), lambda i: (i, 0)),
            pl.BlockSpec((_BLK, 16), lambda i: (i, 0)),
            pl.BlockSpec((_BLK, 16), lambda i: (i, 0)),
            pl.BlockSpec((1, 16), lambda i: (0, 0)),
        ],
        out_shape=[
            jax.ShapeDtypeStruct((2, N, _R0), jnp.float32),
            jax.ShapeDtypeStruct((N, HEADS * HID), jnp.float32),
            jax.ShapeDtypeStruct((N, 16), jnp.float32),
            jax.ShapeDtypeStruct((N, 16), jnp.float32),
            jax.ShapeDtypeStruct((1, 16), jnp.float32),
        ],
    )(x, W_in, b_in, fc_W0, res_W0, al0, ar0)


def _mid_body(Sp_ref, res_ref, bias_ref, lng_ref, lnb_ref,
              fcW1_ref, al1_ref, ar1_ref,
              F_ref, elp_ref, erp_ref, gm_ref):
    i = pl.program_id(0)
    Sa = Sp_ref[0]
    Sb = Sp_ref[1]
    den = jnp.concatenate([Sa[:, 2 * HID:2 * HID + 2],
                           Sb[:, 2 * HID:2 * HID + 2]], axis=1)
    inv = 1.0 / jnp.maximum(den, 1e-9)
    inv = jnp.repeat(inv, HID, axis=1)
    S = jnp.concatenate([Sa[:, :2 * HID], Sb[:, :2 * HID]], axis=1)
    rst = S * inv + res_ref[...] + bias_ref[...]
    mu = jnp.mean(rst, axis=-1, keepdims=True)
    var = jnp.mean((rst - mu) ** 2, axis=-1, keepdims=True)
    hn = (rst - mu) / jnp.sqrt(var + 1e-5) * lng_ref[...] + lnb_ref[...]
    h = jnp.where(hn > 0, hn, jnp.exp(jnp.minimum(hn, 0.0)) - 1.0)
    feat = jnp.dot(h, fcW1_ref[...], preferred_element_type=jnp.float32)
    el = jnp.sum(feat * al1_ref[...], axis=-1, keepdims=True)
    er = jnp.sum(feat * ar1_ref[...], axis=-1, keepdims=True)
    zpad = jnp.zeros((_BLK, 15), jnp.float32)
    elp_ref[...] = jnp.concatenate([el, zpad], axis=1)
    erp_ref[...] = jnp.concatenate([er, zpad], axis=1)
    @pl.when(i == 0)
    def _():
        gm_ref[...] = jnp.full((1, 16), -1e30, jnp.float32)
    gm_ref[...] = jnp.maximum(gm_ref[...], jnp.max(el))
    ones = jnp.ones((_BLK, 1), jnp.float32)
    fpad = jnp.zeros((_BLK, _R1 - OUT - 1), jnp.float32)
    F_ref[...] = jnp.concatenate([feat, ones, fpad], axis=1)


def _mid(S0p, res0, bias0, ln_g, ln_b, fc_W1, al1, ar1):
    return pl.pallas_call(
        _mid_body,
        grid=(N // _BLK,),
        in_specs=[
            pl.BlockSpec((2, _BLK, _R0), lambda i: (0, i, 0)),
            pl.BlockSpec((_BLK, HEADS * HID), lambda i: (i, 0)),
            pl.BlockSpec((1, HEADS * HID), lambda i: (0, 0)),
            pl.BlockSpec((1, HEADS * HID), lambda i: (0, 0)),
            pl.BlockSpec((1, HEADS * HID), lambda i: (0, 0)),
            pl.BlockSpec((HEADS * HID, OUT), lambda i: (0, 0)),
            pl.BlockSpec((1, OUT), lambda i: (0, 0)),
            pl.BlockSpec((1, OUT), lambda i: (0, 0)),
        ],
        out_specs=[
            pl.BlockSpec((_BLK, _R1), lambda i: (i, 0)),
            pl.BlockSpec((_BLK, 16), lambda i: (i, 0)),
            pl.BlockSpec((_BLK, 16), lambda i: (i, 0)),
            pl.BlockSpec((1, 16), lambda i: (0, 0)),
        ],
        out_shape=[
            jax.ShapeDtypeStruct((N, _R1), jnp.float32),
            jax.ShapeDtypeStruct((N, 16), jnp.float32),
            jax.ShapeDtypeStruct((N, 16), jnp.float32),
            jax.ShapeDtypeStruct((1, 16), jnp.float32),
        ],
    )(S0p, res0, bias0, ln_g, ln_b, fc_W1, al1, ar1)


def _fin_body(Sp_ref, bias_ref, out_ref):
    agg = Sp_ref[0] + Sp_ref[1]
    den = jnp.maximum(agg[:, OUT:OUT + 1], 1e-9)
    out_ref[...] = agg[:, :OUT] / den + bias_ref[...]


def _fin(S1p, bias1):
    return pl.pallas_call(
        _fin_body,
        grid=(N // _BLK,),
        in_specs=[
            pl.BlockSpec((2, _BLK, _R1), lambda i: (0, i, 0)),
            pl.BlockSpec((1, OUT), lambda i: (0, 0)),
        ],
        out_specs=pl.BlockSpec((_BLK, OUT), lambda i: (i, 0)),
        out_shape=jax.ShapeDtypeStruct((N, OUT), jnp.float32),
    )(S1p, bias1)


def _sc_mesh():
    return plsc.VectorSubcoreMesh(core_axis_name="c", subcore_axis_name="s")


_SC_PARAMS = pltpu.CompilerParams(use_tc_tiling_on_sc=False)


def _agg_kernel(r_width, heads_split, ch):
    nv = r_width // 16
    ng = ch // 16
    if heads_split:
        total_chunks = E // ch
    else:
        total_chunks = (E // 2) // ch
    nl = (total_chunks + 15) // 16
    nt2 = (nl + 1) // 2

    def body(src_hbm, dst_hbm, elp_hbm, erp_hbm, gm_hbm, F_hbm, out_hbm,
             rows_v, el_v, er_v, src_v, dst_v, sdst_v, idx2, gm_v, S_sh,
             sl0, sl1, sg0, sg1, ss0, ss1):
        c = lax.axis_index("c")
        s = lax.axis_index("s")
        base = s * nl
        cnt = jnp.minimum(nl, total_chunks - s * nl)
        eoff0 = (0 if heads_split else c * (E // 2)) + base * ch
        sl = (sl0, sl1)
        sg = (sg0, sg1)
        ss = (ss0, ss1)

        def zrow(j, _):
            for v in range(nv):
                rows_v[0, j, pl.ds(v * 16, 16)] = jnp.zeros((16,), jnp.float32)
            return 0

        lax.fori_loop(0, ch, zrow, 0)
        for q in range(_RPT // ch):
            pltpu.sync_copy(rows_v.at[0],
                            S_sh.at[pl.ds(s * _RPT + q * ch, ch), :])
        plsc.subcore_barrier()

        pltpu.sync_copy(gm_hbm, gm_v)
        gs = gm_v[0, :]

        def fire_lin(local, k):
            off = eoff0 + local * ch
            pltpu.async_copy(src_hbm.at[pl.ds(off, ch)], src_v.at[k], sl[k])
            pltpu.async_copy(dst_hbm.at[pl.ds(off, ch)], dst_v.at[k], sl[k])

        def drain_lin(k):
            pltpu.make_async_copy(src_hbm.at[pl.ds(0, ch)],
                                  src_v.at[k], sl[k]).wait()
            pltpu.make_async_copy(dst_hbm.at[pl.ds(0, ch)],
                                  dst_v.at[k], sl[k]).wait()

        def fire_gather(k):
            if heads_split:
                for g in range(ng):
                    sv = src_v[k, pl.ds(g * 16, 16)]
                    idx2[k, pl.ds(g * 16, 16)] = sv + c * N
                fidx = idx2.at[k]
            else:
                fidx = src_v.at[k]
            pltpu.async_copy(F_hbm.at[fidx], rows_v.at[k], sg[k])
            pltpu.async_copy(elp_hbm.at[src_v.at[k]], el_v.at[k], sg[k])
            pltpu.async_copy(erp_hbm.at[dst_v.at[k]], er_v.at[k], sg[k])

        def drain_gather(k):
            pltpu.make_async_copy(F_hbm.at[pl.ds(0, ch), :],
                                  rows_v.at[k], sg[k]).wait()
            pltpu.make_async_copy(elp_hbm.at[pl.ds(0, ch), :],
                                  el_v.at[k], sg[k]).wait()
            pltpu.make_async_copy(erp_hbm.at[pl.ds(0, ch), :],
                                  er_v.at[k], sg[k]).wait()

        def drain_scatter(k):
            pltpu.make_async_copy(rows_v.at[k], S_sh.at[sdst_v.at[k]],
                                  ss[k]).wait()

        def scale(k):
            @plsc.parallel_loop(0, ch)
            def _(j):
                el = el_v[k, j, :]
                er = er_v[k, j, :]
                ee = jnp.exp(_lrelu(el + er) - _lrelu(gs + er))
                if heads_split:
                    m0s = jnp.where(c == 0, ee[0], ee[2])
                    m1s = jnp.where(c == 0, ee[1], ee[3])
                else:
                    m0s = ee[0]
                    m1s = ee[0]
                m0 = jnp.full((16,), m0s, jnp.float32)
                m1 = jnp.full((16,), m1s, jnp.float32)
                half = (nv - 1) // 2 if heads_split else nv - 1
                for w in range(nv - 1):
                    r = rows_v[k, j, pl.ds(w * 16, 16)]
                    rows_v[k, j, pl.ds(w * 16, 16)] = r * (m0 if w < half
                                                           else m1)
                io = lax.broadcasted_iota(jnp.int32, (16,), 0)
                mult = jnp.where(io == 0, m0, jnp.where(io == 1, m1, 0.0))
                r = rows_v[k, j, pl.ds((nv - 1) * 16, 16)]
                rows_v[k, j, pl.ds((nv - 1) * 16, 16)] = r * mult

        @pl.when(cnt > 0)
        def _():
            fire_lin(0, 0)

        @pl.when(cnt > 1)
        def _():
            fire_lin(1, 1)

        @pl.when(cnt > 0)
        def _():
            drain_lin(0)
            fire_gather(0)

        def it(t, _):
            for k in (0, 1):
                local = t * 2 + k

                @pl.when((local >= 1) & (local < cnt))
                def _():
                    drain_scatter(1 - k)

                @pl.when(local + 1 < cnt)
                def _():
                    drain_lin(1 - k)
                    fire_gather(1 - k)

                @pl.when(local < cnt)
                def _():
                    drain_gather(k)
                    for g in range(ng):
                        sdst_v[k, pl.ds(g * 16, 16)] = \
                            dst_v[k, pl.ds(g * 16, 16)]

                @pl.when(local + 2 < cnt)
                def _():
                    fire_lin(local + 2, k)

                @pl.when(local < cnt)
                def _():
                    scale(k)
                    pltpu.async_copy(rows_v.at[k], S_sh.at[sdst_v.at[k]],
                                     ss[k], add=True)
            return 0

        lax.fori_loop(0, nt2, it, 0)
        for k in (0, 1):
            @pl.when((cnt >= 1) & (lax.rem(cnt - 1, 2) == k))
            def _():
                drain_scatter(k)
        plsc.subcore_barrier()
        pltpu.sync_copy(S_sh.at[pl.ds(s * _RPT, _RPT), :],
                        out_hbm.at[c, pl.ds(s * _RPT, _RPT), :])

    kern = functools.partial(
        pl.kernel, mesh=_sc_mesh(),
        out_type=jax.ShapeDtypeStruct((2, NP, r_width), jnp.float32),
        compiler_params=_SC_PARAMS,
        scratch_types=[
            pltpu.VMEM((2, ch, r_width), jnp.float32),
            pltpu.VMEM((2, ch, 16), jnp.float32),
            pltpu.VMEM((2, ch, 16), jnp.float32),
            pltpu.VMEM((2, ch), jnp.int32),
            pltpu.VMEM((2, ch), jnp.int32),
            pltpu.VMEM((2, ch), jnp.int32),
            pltpu.VMEM((2, ch), jnp.int32),
            pltpu.VMEM((1, 16), jnp.float32),
            pltpu.VMEM_SHARED((NP, r_width), jnp.float32),
            pltpu.SemaphoreType.DMA,
            pltpu.SemaphoreType.DMA,
            pltpu.SemaphoreType.DMA,
            pltpu.SemaphoreType.DMA,
            pltpu.SemaphoreType.DMA,
            pltpu.SemaphoreType.DMA,
        ],
    )
    return kern(body)


def kernel(x, edge_index0, edge_index1, W_in, b_in, fc_W0, bias0, attn_l0,
           attn_r0, res_W0, ln_g, ln_b, fc_W1, bias1, attn_l1, attn_r1):
    src0, dst0 = edge_index0[0], edge_index0[1]
    src1, dst1 = edge_index1[0], edge_index1[1]

    F0, res0, elp0, erp0, gm0 = _proj0(x, W_in, b_in, fc_W0, res_W0,
                                       attn_l0, attn_r0)
    S0 = _agg_kernel(_R0, True, 80)(src0, dst0, elp0, erp0, gm0,
                                    F0.reshape(2 * N, _R0))
    F1, elp1, erp1, gm1 = _mid(S0, res0, bias0.reshape(1, HEADS * HID),
                               ln_g.reshape(1, -1), ln_b.reshape(1, -1),
                               fc_W1, attn_l1, attn_r1)
    S1 = _agg_kernel(_R1, False, 128)(src1, dst1, elp1, erp1, gm1, F1)
    out = _fin(S1, bias1.reshape(1, OUT))
    return out


# speedup vs baseline: 81.6860x; 1.0540x over previous
import functools

import jax
import jax.numpy as jnp
from jax import lax
from jax.experimental import pallas as pl
from jax.experimental.pallas import tpu as pltpu
import jax.experimental.pallas.tpu_sc as plsc

N = 10000
NP = 10240
E = 320000
D_IN = 128
HID = 64
HEADS = 4
OUT = 64
NEG_SLOPE = 0.2

_BLK = 400
_R0 = 144
_R1 = 80
_NB = E // 128
_EPAD = 2560
_RPT = NP // 16


def _lrelu(x):
    return jnp.where(x > 0, x, NEG_SLOPE * x)


def _proj0_body(x_ref, Win_ref, bin_ref, fcW_ref, resW_ref, al_ref, ar_ref,
                F_ref, res_ref, erp_ref, gm_ref):
    i = pl.program_id(0)
    x = x_ref[...]
    h = jnp.dot(x, Win_ref[...], preferred_element_type=jnp.float32) + bin_ref[...]
    feat = jnp.dot(h, fcW_ref[...], preferred_element_type=jnp.float32)
    res_ref[...] = jnp.dot(h, resW_ref[...], preferred_element_type=jnp.float32)
    f = feat.reshape(_BLK, HEADS, HID)
    el = jnp.sum(f * al_ref[...][None], axis=-1)
    er = jnp.sum(f * ar_ref[...][None], axis=-1)
    zpad = jnp.zeros((_BLK, 16 - HEADS), jnp.float32)
    erp_ref[...] = jnp.concatenate([er, zpad], axis=1)
    @pl.when(i == 0)
    def _():
        gm_ref[...] = jnp.full((1, 16), -1e30, jnp.float32)
    gm_ref[...] = jnp.maximum(gm_ref[...], jnp.max(el))
    ones = jnp.ones((_BLK, 2), jnp.float32)
    fpad = jnp.zeros((_BLK, _R0 - 2 * HID - HEADS - 2), jnp.float32)
    F_ref[0] = jnp.concatenate([feat[:, :2 * HID], el, ones, fpad], axis=1)
    F_ref[1] = jnp.concatenate([feat[:, 2 * HID:], el, ones, fpad], axis=1)


def _proj0(x, W_in, b_in, fc_W0, res_W0, al0, ar0):
    return pl.pallas_call(
        _proj0_body,
        grid=(N // _BLK,),
        in_specs=[
            pl.BlockSpec((_BLK, D_IN), lambda i: (i, 0)),
            pl.BlockSpec((D_IN, HID), lambda i: (0, 0)),
            pl.BlockSpec((HID,), lambda i: (0,)),
            pl.BlockSpec((HID, HEADS * HID), lambda i: (0, 0)),
            pl.BlockSpec((HID, HEADS * HID), lambda i: (0, 0)),
            pl.BlockSpec((HEADS, HID), lambda i: (0, 0)),
            pl.BlockSpec((HEADS, HID), lambda i: (0, 0)),
        ],
        out_specs=[
            pl.BlockSpec((2, _BLK, _R0), lambda i: (0, i, 0)),
            pl.BlockSpec((_BLK, HEADS * HID), lambda i: (i, 0)),
            pl.BlockSpec((_BLK, 16), lambda i: (i, 0)),
            pl.BlockSpec((1, 16), lambda i: (0, 0)),
        ],
        out_shape=[
            jax.ShapeDtypeStruct((2, N, _R0), jnp.float32),
            jax.ShapeDtypeStruct((N, HEADS * HID), jnp.float32),
            jax.ShapeDtypeStruct((N, 16), jnp.float32),
            jax.ShapeDtypeStruct((1, 16), jnp.float32),
        ],
    )(x, W_in, b_in, fc_W0, res_W0, al0, ar0)


def _mid_body(Sp_ref, res_ref, bias_ref, lng_ref, lnb_ref,
              fcW1_ref, al1_ref, ar1_ref,
              F_ref, erp_ref, gm_ref):
    i = pl.program_id(0)
    Sa = Sp_ref[0]
    Sb = Sp_ref[1]
    dcol = 2 * HID + HEADS
    den = jnp.concatenate([Sa[:, dcol:dcol + 2],
                           Sb[:, dcol:dcol + 2]], axis=1)
    inv = 1.0 / jnp.maximum(den, 1e-9)
    inv = jnp.repeat(inv, HID, axis=1)
    S = jnp.concatenate([Sa[:, :2 * HID], Sb[:, :2 * HID]], axis=1)
    rst = S * inv + res_ref[...] + bias_ref[...]
    mu = jnp.mean(rst, axis=-1, keepdims=True)
    var = jnp.mean((rst - mu) ** 2, axis=-1, keepdims=True)
    hn = (rst - mu) / jnp.sqrt(var + 1e-5) * lng_ref[...] + lnb_ref[...]
    h = jnp.where(hn > 0, hn, jnp.exp(jnp.minimum(hn, 0.0)) - 1.0)
    feat = jnp.dot(h, fcW1_ref[...], preferred_element_type=jnp.float32)
    el = jnp.sum(feat * al1_ref[...], axis=-1, keepdims=True)
    er = jnp.sum(feat * ar1_ref[...], axis=-1, keepdims=True)
    zpad = jnp.zeros((_BLK, 15), jnp.float32)
    erp_ref[...] = jnp.concatenate([er, zpad], axis=1)
    @pl.when(i == 0)
    def _():
        gm_ref[...] = jnp.full((1, 16), -1e30, jnp.float32)
    gm_ref[...] = jnp.maximum(gm_ref[...], jnp.max(el))
    ones = jnp.ones((_BLK, 1), jnp.float32)
    fpad = jnp.zeros((_BLK, _R1 - OUT - 2), jnp.float32)
    F_ref[...] = jnp.concatenate([feat, el, ones, fpad], axis=1)


def _mid(S0p, res0, bias0, ln_g, ln_b, fc_W1, al1, ar1):
    return pl.pallas_call(
        _mid_body,
        grid=(N // _BLK,),
        in_specs=[
            pl.BlockSpec((2, _BLK, _R0), lambda i: (0, i, 0)),
            pl.BlockSpec((_BLK, HEADS * HID), lambda i: (i, 0)),
            pl.BlockSpec((1, HEADS * HID), lambda i: (0, 0)),
            pl.BlockSpec((1, HEADS * HID), lambda i: (0, 0)),
            pl.BlockSpec((1, HEADS * HID), lambda i: (0, 0)),
            pl.BlockSpec((HEADS * HID, OUT), lambda i: (0, 0)),
            pl.BlockSpec((1, OUT), lambda i: (0, 0)),
            pl.BlockSpec((1, OUT), lambda i: (0, 0)),
        ],
        out_specs=[
            pl.BlockSpec((_BLK, _R1), lambda i: (i, 0)),
            pl.BlockSpec((_BLK, 16), lambda i: (i, 0)),
            pl.BlockSpec((1, 16), lambda i: (0, 0)),
        ],
        out_shape=[
            jax.ShapeDtypeStruct((N, _R1), jnp.float32),
            jax.ShapeDtypeStruct((N, 16), jnp.float32),
            jax.ShapeDtypeStruct((1, 16), jnp.float32),
        ],
    )(S0p, res0, bias0, ln_g, ln_b, fc_W1, al1, ar1)


def _fin_body(Sp_ref, bias_ref, out_ref):
    agg = Sp_ref[0] + Sp_ref[1]
    den = jnp.maximum(agg[:, OUT + 1:OUT + 2], 1e-9)
    out_ref[...] = agg[:, :OUT] / den + bias_ref[...]


def _fin(S1p, bias1):
    return pl.pallas_call(
        _fin_body,
        grid=(N // _BLK,),
        in_specs=[
            pl.BlockSpec((2, _BLK, _R1), lambda i: (0, i, 0)),
            pl.BlockSpec((1, OUT), lambda i: (0, 0)),
        ],
        out_specs=pl.BlockSpec((_BLK, OUT), lambda i: (i, 0)),
        out_shape=jax.ShapeDtypeStruct((N, OUT), jnp.float32),
    )(S1p, bias1)


def _sc_mesh():
    return plsc.VectorSubcoreMesh(core_axis_name="c", subcore_axis_name="s")


_SC_PARAMS = pltpu.CompilerParams(use_tc_tiling_on_sc=False)


def _agg_kernel(r_width, heads_split, ch):
    nv = r_width // 16
    ng = ch // 16
    if heads_split:
        total_chunks = E // ch
    else:
        total_chunks = (E // 2) // ch
    nl = (total_chunks + 15) // 16
    nt2 = (nl + 1) // 2

    def body(src_hbm, dst_hbm, erp_hbm, gm_hbm, F_hbm, out_hbm,
             rows_v, er_v, src_v, dst_v, sdst_v, idx2, gm_v, S_sh,
             sl0, sl1, sg0, sg1, ss0, ss1):
        c = lax.axis_index("c")
        s = lax.axis_index("s")
        base = s * nl
        cnt = jnp.minimum(nl, total_chunks - s * nl)
        eoff0 = (0 if heads_split else c * (E // 2)) + base * ch
        sl = (sl0, sl1)
        sg = (sg0, sg1)
        ss = (ss0, ss1)

        def zrow(j, _):
            for v in range(nv):
                rows_v[0, j, pl.ds(v * 16, 16)] = jnp.zeros((16,), jnp.float32)
            return 0

        lax.fori_loop(0, ch, zrow, 0)
        for q in range(_RPT // ch):
            pltpu.sync_copy(rows_v.at[0],
                            S_sh.at[pl.ds(s * _RPT + q * ch, ch), :])
        plsc.subcore_barrier()

        pltpu.sync_copy(gm_hbm, gm_v)
        gs = gm_v[0, :]

        def fire_lin(local, k):
            off = eoff0 + local * ch
            pltpu.async_copy(src_hbm.at[pl.ds(off, ch)], src_v.at[k], sl[k])
            pltpu.async_copy(dst_hbm.at[pl.ds(off, ch)], dst_v.at[k], sl[k])

        def drain_lin(k):
            pltpu.make_async_copy(src_hbm.at[pl.ds(0, ch)],
                                  src_v.at[k], sl[k]).wait()
            pltpu.make_async_copy(dst_hbm.at[pl.ds(0, ch)],
                                  dst_v.at[k], sl[k]).wait()

        def fire_gather(k):
            if heads_split:
                for g in range(ng):
                    sv = src_v[k, pl.ds(g * 16, 16)]
                    idx2[k, pl.ds(g * 16, 16)] = sv + c * N
                fidx = idx2.at[k]
            else:
                fidx = src_v.at[k]
            pltpu.async_copy(F_hbm.at[fidx], rows_v.at[k], sg[k])
            pltpu.async_copy(erp_hbm.at[dst_v.at[k]], er_v.at[k], sg[k])

        def drain_gather(k):
            pltpu.make_async_copy(F_hbm.at[pl.ds(0, ch), :],
                                  rows_v.at[k], sg[k]).wait()
            pltpu.make_async_copy(erp_hbm.at[pl.ds(0, ch), :],
                                  er_v.at[k], sg[k]).wait()

        def drain_scatter(k):
            pltpu.make_async_copy(rows_v.at[k], S_sh.at[sdst_v.at[k]],
                                  ss[k]).wait()

        def scale(k):
            @plsc.parallel_loop(0, ch)
            def _(j):
                el = rows_v[k, j, pl.ds((nv - 1) * 16, 16)]
                er = er_v[k, j, :]
                ee = jnp.exp(_lrelu(el + er) - _lrelu(gs + er))
                if heads_split:
                    m0s = jnp.where(c == 0, ee[0], ee[2])
                    m1s = jnp.where(c == 0, ee[1], ee[3])
                    dl0, dl1 = HEADS, HEADS + 1
                else:
                    m0s = ee[0]
                    m1s = ee[0]
                    dl0, dl1 = 1, 1
                m0 = jnp.full((16,), m0s, jnp.float32)
                m1 = jnp.full((16,), m1s, jnp.float32)
                half = (nv - 1) // 2 if heads_split else nv - 1
                for w in range(nv - 1):
                    r = rows_v[k, j, pl.ds(w * 16, 16)]
                    rows_v[k, j, pl.ds(w * 16, 16)] = r * (m0 if w < half
                                                           else m1)
                io = lax.broadcasted_iota(jnp.int32, (16,), 0)
                mult = jnp.where(io == dl0, m0, jnp.where(io == dl1, m1, 0.0))
                rows_v[k, j, pl.ds((nv - 1) * 16, 16)] = el * mult

        @pl.when(cnt > 0)
        def _():
            fire_lin(0, 0)

        @pl.when(cnt > 1)
        def _():
            fire_lin(1, 1)

        @pl.when(cnt > 0)
        def _():
            drain_lin(0)
            fire_gather(0)

        def it(t, _):
            for k in (0, 1):
                local = t * 2 + k

                @pl.when((local >= 1) & (local < cnt))
                def _():
                    drain_scatter(1 - k)

                @pl.when(local + 1 < cnt)
                def _():
                    drain_lin(1 - k)
                    fire_gather(1 - k)

                @pl.when(local < cnt)
                def _():
                    drain_gather(k)
                    for g in range(ng):
                        sdst_v[k, pl.ds(g * 16, 16)] = \
                            dst_v[k, pl.ds(g * 16, 16)]

                @pl.when(local + 2 < cnt)
                def _():
                    fire_lin(local + 2, k)

                @pl.when(local < cnt)
                def _():
                    scale(k)
                    pltpu.async_copy(rows_v.at[k], S_sh.at[sdst_v.at[k]],
                                     ss[k], add=True)
            return 0

        lax.fori_loop(0, nt2, it, 0)
        for k in (0, 1):
            @pl.when((cnt >= 1) & (lax.rem(cnt - 1, 2) == k))
            def _():
                drain_scatter(k)
        plsc.subcore_barrier()
        pltpu.sync_copy(S_sh.at[pl.ds(s * _RPT, _RPT), :],
                        out_hbm.at[c, pl.ds(s * _RPT, _RPT), :])

    kern = functools.partial(
        pl.kernel, mesh=_sc_mesh(),
        out_type=jax.ShapeDtypeStruct((2, NP, r_width), jnp.float32),
        compiler_params=_SC_PARAMS,
        scratch_types=[
            pltpu.VMEM((2, ch, r_width), jnp.float32),
            pltpu.VMEM((2, ch, 16), jnp.float32),
            pltpu.VMEM((2, ch), jnp.int32),
            pltpu.VMEM((2, ch), jnp.int32),
            pltpu.VMEM((2, ch), jnp.int32),
            pltpu.VMEM((2, ch), jnp.int32),
            pltpu.VMEM((1, 16), jnp.float32),
            pltpu.VMEM_SHARED((NP, r_width), jnp.float32),
            pltpu.SemaphoreType.DMA,
            pltpu.SemaphoreType.DMA,
            pltpu.SemaphoreType.DMA,
            pltpu.SemaphoreType.DMA,
            pltpu.SemaphoreType.DMA,
            pltpu.SemaphoreType.DMA,
        ],
    )
    return kern(body)


def kernel(x, edge_index0, edge_index1, W_in, b_in, fc_W0, bias0, attn_l0,
           attn_r0, res_W0, ln_g, ln_b, fc_W1, bias1, attn_l1, attn_r1):
    src0, dst0 = edge_index0[0], edge_index0[1]
    src1, dst1 = edge_index1[0], edge_index1[1]

    F0, res0, erp0, gm0 = _proj0(x, W_in, b_in, fc_W0, res_W0,
                                 attn_l0, attn_r0)
    S0 = _agg_kernel(_R0, True, 80)(src0, dst0, erp0, gm0,
                                    F0.reshape(2 * N, _R0))
    F1, erp1, gm1 = _mid(S0, res0, bias0.reshape(1, HEADS * HID),
                         ln_g.reshape(1, -1), ln_b.reshape(1, -1),
                         fc_W1, attn_l1, attn_r1)
    S1 = _agg_kernel(_R1, False, 128)(src1, dst1, erp1, gm1, F1)
    out = _fin(S1, bias1.reshape(1, OUT))
    return out


# speedup vs baseline: 82.6400x; 1.0117x over previous
import functools

import jax
import jax.numpy as jnp
from jax import lax
from jax.experimental import pallas as pl
from jax.experimental.pallas import tpu as pltpu
import jax.experimental.pallas.tpu_sc as plsc

N = 10000
NP = 10240
E = 320000
D_IN = 128
HID = 64
HEADS = 4
OUT = 64
NEG_SLOPE = 0.2

_BLK = 400
_R0 = 144
_R1 = 80
_NB = E // 128
_EPAD = 2560
_RPT = NP // 16


def _lrelu(x):
    return jnp.where(x > 0, x, NEG_SLOPE * x)


def _proj0_body(x_ref, Win_ref, bin_ref, fcW_ref, resW_ref, al_ref, ar_ref,
                F_ref, res_ref, erp_ref, gm_ref):
    i = pl.program_id(0)
    x = x_ref[...]
    h = jnp.dot(x, Win_ref[...], preferred_element_type=jnp.float32) + bin_ref[...]
    feat = jnp.dot(h, fcW_ref[...], preferred_element_type=jnp.float32)
    res_ref[...] = jnp.dot(h, resW_ref[...], preferred_element_type=jnp.float32)
    f = feat.reshape(_BLK, HEADS, HID)
    el = jnp.sum(f * al_ref[...][None], axis=-1)
    er = jnp.sum(f * ar_ref[...][None], axis=-1)
    zpad = jnp.zeros((_BLK, 16 - HEADS), jnp.float32)
    erp_ref[...] = jnp.concatenate([er, zpad], axis=1)
    @pl.when(i == 0)
    def _():
        gm_ref[...] = jnp.full((1, 16), -1e30, jnp.float32)
    gm_ref[...] = jnp.maximum(gm_ref[...], jnp.max(el))
    ones = jnp.ones((_BLK, 2), jnp.float32)
    fpad = jnp.zeros((_BLK, _R0 - 2 * HID - HEADS - 2), jnp.float32)
    F_ref[0] = jnp.concatenate([feat[:, :2 * HID], el, ones, fpad], axis=1)
    F_ref[1] = jnp.concatenate([feat[:, 2 * HID:], el, ones, fpad], axis=1)


def _proj0(x, W_in, b_in, fc_W0, res_W0, al0, ar0):
    return pl.pallas_call(
        _proj0_body,
        grid=(N // _BLK,),
        in_specs=[
            pl.BlockSpec((_BLK, D_IN), lambda i: (i, 0)),
            pl.BlockSpec((D_IN, HID), lambda i: (0, 0)),
            pl.BlockSpec((HID,), lambda i: (0,)),
            pl.BlockSpec((HID, HEADS * HID), lambda i: (0, 0)),
            pl.BlockSpec((HID, HEADS * HID), lambda i: (0, 0)),
            pl.BlockSpec((HEADS, HID), lambda i: (0, 0)),
            pl.BlockSpec((HEADS, HID), lambda i: (0, 0)),
        ],
        out_specs=[
            pl.BlockSpec((2, _BLK, _R0), lambda i: (0, i, 0)),
            pl.BlockSpec((_BLK, HEADS * HID), lambda i: (i, 0)),
            pl.BlockSpec((_BLK, 16), lambda i: (i, 0)),
            pl.BlockSpec((1, 16), lambda i: (0, 0)),
        ],
        out_shape=[
            jax.ShapeDtypeStruct((2, N, _R0), jnp.float32),
            jax.ShapeDtypeStruct((N, HEADS * HID), jnp.float32),
            jax.ShapeDtypeStruct((N, 16), jnp.float32),
            jax.ShapeDtypeStruct((1, 16), jnp.float32),
        ],
    )(x, W_in, b_in, fc_W0, res_W0, al0, ar0)


def _mid_body(Sp_ref, res_ref, bias_ref, lng_ref, lnb_ref,
              fcW1_ref, al1_ref, ar1_ref,
              F_ref, erp_ref, gm_ref):
    i = pl.program_id(0)
    Sa = Sp_ref[0]
    Sb = Sp_ref[1]
    dcol = 2 * HID + HEADS
    den = jnp.concatenate([Sa[:, dcol:dcol + 2],
                           Sb[:, dcol:dcol + 2]], axis=1)
    inv = 1.0 / jnp.maximum(den, 1e-9)
    inv = jnp.repeat(inv, HID, axis=1)
    S = jnp.concatenate([Sa[:, :2 * HID], Sb[:, :2 * HID]], axis=1)
    rst = S * inv + res_ref[...] + bias_ref[...]
    mu = jnp.mean(rst, axis=-1, keepdims=True)
    var = jnp.mean((rst - mu) ** 2, axis=-1, keepdims=True)
    hn = (rst - mu) / jnp.sqrt(var + 1e-5) * lng_ref[...] + lnb_ref[...]
    h = jnp.where(hn > 0, hn, jnp.exp(jnp.minimum(hn, 0.0)) - 1.0)
    feat = jnp.dot(h, fcW1_ref[...], preferred_element_type=jnp.float32)
    el = jnp.sum(feat * al1_ref[...], axis=-1, keepdims=True)
    er = jnp.sum(feat * ar1_ref[...], axis=-1, keepdims=True)
    zpad = jnp.zeros((_BLK, 15), jnp.float32)
    erp_ref[...] = jnp.concatenate([er, zpad], axis=1)
    @pl.when(i == 0)
    def _():
        gm_ref[...] = jnp.full((1, 16), -1e30, jnp.float32)
    gm_ref[...] = jnp.maximum(gm_ref[...], jnp.max(el))
    ones = jnp.ones((_BLK, 1), jnp.float32)
    fpad = jnp.zeros((_BLK, _R1 - OUT - 2), jnp.float32)
    F_ref[...] = jnp.concatenate([feat, el, ones, fpad], axis=1)


def _mid(S0p, res0, bias0, ln_g, ln_b, fc_W1, al1, ar1):
    return pl.pallas_call(
        _mid_body,
        grid=(N // _BLK,),
        in_specs=[
            pl.BlockSpec((2, _BLK, _R0), lambda i: (0, i, 0)),
            pl.BlockSpec((_BLK, HEADS * HID), lambda i: (i, 0)),
            pl.BlockSpec((1, HEADS * HID), lambda i: (0, 0)),
            pl.BlockSpec((1, HEADS * HID), lambda i: (0, 0)),
            pl.BlockSpec((1, HEADS * HID), lambda i: (0, 0)),
            pl.BlockSpec((HEADS * HID, OUT), lambda i: (0, 0)),
            pl.BlockSpec((1, OUT), lambda i: (0, 0)),
            pl.BlockSpec((1, OUT), lambda i: (0, 0)),
        ],
        out_specs=[
            pl.BlockSpec((_BLK, _R1), lambda i: (i, 0)),
            pl.BlockSpec((_BLK, 16), lambda i: (i, 0)),
            pl.BlockSpec((1, 16), lambda i: (0, 0)),
        ],
        out_shape=[
            jax.ShapeDtypeStruct((N, _R1), jnp.float32),
            jax.ShapeDtypeStruct((N, 16), jnp.float32),
            jax.ShapeDtypeStruct((1, 16), jnp.float32),
        ],
    )(S0p, res0, bias0, ln_g, ln_b, fc_W1, al1, ar1)


def _fin_body(Sp_ref, bias_ref, out_ref):
    agg = Sp_ref[0] + Sp_ref[1]
    den = jnp.maximum(agg[:, OUT + 1:OUT + 2], 1e-9)
    out_ref[...] = agg[:, :OUT] / den + bias_ref[...]


def _fin(S1p, bias1):
    return pl.pallas_call(
        _fin_body,
        grid=(N // _BLK,),
        in_specs=[
            pl.BlockSpec((2, _BLK, _R1), lambda i: (0, i, 0)),
            pl.BlockSpec((1, OUT), lambda i: (0, 0)),
        ],
        out_specs=pl.BlockSpec((_BLK, OUT), lambda i: (i, 0)),
        out_shape=jax.ShapeDtypeStruct((N, OUT), jnp.float32),
    )(S1p, bias1)


def _sc_mesh():
    return plsc.VectorSubcoreMesh(core_axis_name="c", subcore_axis_name="s")


_SC_PARAMS = pltpu.CompilerParams(use_tc_tiling_on_sc=False)


def _agg_kernel(r_width, heads_split, ch):
    nv = r_width // 16
    ng = ch // 16
    if heads_split:
        total_chunks = E // ch
    else:
        total_chunks = (E // 2) // ch
    nl = (total_chunks + 15) // 16
    nt2 = (nl + 1) // 2

    def body(src_hbm, dst_hbm, erp_hbm, gm_hbm, F_hbm, out_hbm,
             rows_v, er_v, src_v, dst_v, sdst_v, idx2, gm_v, S_sh,
             sl0, sl1, sg0, sg1, ss0, ss1):
        c = lax.axis_index("c")
        s = lax.axis_index("s")
        base = s * nl
        cnt = jnp.minimum(nl, total_chunks - s * nl)
        eoff0 = (0 if heads_split else c * (E // 2)) + base * ch
        sl = (sl0, sl1)
        sg = (sg0, sg1)
        ss = (ss0, ss1)

        def zrow(j, _):
            for v in range(nv):
                rows_v[0, j, pl.ds(v * 16, 16)] = jnp.zeros((16,), jnp.float32)
            return 0

        lax.fori_loop(0, ch, zrow, 0)
        for q in range(_RPT // ch):
            pltpu.sync_copy(rows_v.at[0],
                            S_sh.at[pl.ds(s * _RPT + q * ch, ch), :])
        plsc.subcore_barrier()

        pltpu.sync_copy(gm_hbm, gm_v)
        gs = gm_v[0, :]

        def fire_lin(local, k):
            off = eoff0 + local * ch
            pltpu.async_copy(src_hbm.at[pl.ds(off, ch)], src_v.at[k], sl[k])
            pltpu.async_copy(dst_hbm.at[pl.ds(off, ch)], dst_v.at[k], sl[k])

        def drain_lin(k):
            pltpu.make_async_copy(src_hbm.at[pl.ds(0, ch)],
                                  src_v.at[k], sl[k]).wait()
            pltpu.make_async_copy(dst_hbm.at[pl.ds(0, ch)],
                                  dst_v.at[k], sl[k]).wait()

        def fire_gather(k):
            if heads_split:
                for g in range(ng):
                    sv = src_v[k, pl.ds(g * 16, 16)]
                    idx2[k, pl.ds(g * 16, 16)] = sv + c * N
                fidx = idx2.at[k]
            else:
                fidx = src_v.at[k]
            pltpu.async_copy(F_hbm.at[fidx], rows_v.at[k], sg[k])
            pltpu.async_copy(erp_hbm.at[dst_v.at[k]], er_v.at[k], sg[k])

        def drain_gather(k):
            pltpu.make_async_copy(F_hbm.at[pl.ds(0, ch), :],
                                  rows_v.at[k], sg[k]).wait()
            pltpu.make_async_copy(erp_hbm.at[pl.ds(0, ch), :],
                                  er_v.at[k], sg[k]).wait()

        def drain_scatter(k):
            pltpu.make_async_copy(rows_v.at[k], S_sh.at[sdst_v.at[k]],
                                  ss[k]).wait()

        def scale(k):
            @plsc.parallel_loop(0, ch, unroll=2)
            def _(j):
                el = rows_v[k, j, pl.ds((nv - 1) * 16, 16)]
                er = er_v[k, j, :]
                ee = jnp.exp(_lrelu(el + er) - _lrelu(gs + er))
                if heads_split:
                    m0s = jnp.where(c == 0, ee[0], ee[2])
                    m1s = jnp.where(c == 0, ee[1], ee[3])
                    dl0, dl1 = HEADS, HEADS + 1
                else:
                    m0s = ee[0]
                    m1s = ee[0]
                    dl0, dl1 = 1, 1
                m0 = jnp.full((16,), m0s, jnp.float32)
                m1 = jnp.full((16,), m1s, jnp.float32)
                half = (nv - 1) // 2 if heads_split else nv - 1
                for w in range(nv - 1):
                    r = rows_v[k, j, pl.ds(w * 16, 16)]
                    rows_v[k, j, pl.ds(w * 16, 16)] = r * (m0 if w < half
                                                           else m1)
                io = lax.broadcasted_iota(jnp.int32, (16,), 0)
                mult = jnp.where(io == dl0, m0, jnp.where(io == dl1, m1, 0.0))
                rows_v[k, j, pl.ds((nv - 1) * 16, 16)] = el * mult

        @pl.when(cnt > 0)
        def _():
            fire_lin(0, 0)

        @pl.when(cnt > 1)
        def _():
            fire_lin(1, 1)

        @pl.when(cnt > 0)
        def _():
            drain_lin(0)
            fire_gather(0)

        def it(t, _):
            for k in (0, 1):
                local = t * 2 + k

                @pl.when((local >= 1) & (local < cnt))
                def _():
                    drain_scatter(1 - k)

                @pl.when(local + 1 < cnt)
                def _():
                    drain_lin(1 - k)
                    fire_gather(1 - k)

                @pl.when(local < cnt)
                def _():
                    drain_gather(k)
                    for g in range(ng):
                        sdst_v[k, pl.ds(g * 16, 16)] = \
                            dst_v[k, pl.ds(g * 16, 16)]

                @pl.when(local + 2 < cnt)
                def _():
                    fire_lin(local + 2, k)

                @pl.when(local < cnt)
                def _():
                    scale(k)
                    pltpu.async_copy(rows_v.at[k], S_sh.at[sdst_v.at[k]],
                                     ss[k], add=True)
            return 0

        lax.fori_loop(0, nt2, it, 0)
        for k in (0, 1):
            @pl.when((cnt >= 1) & (lax.rem(cnt - 1, 2) == k))
            def _():
                drain_scatter(k)
        plsc.subcore_barrier()
        pltpu.sync_copy(S_sh.at[pl.ds(s * _RPT, _RPT), :],
                        out_hbm.at[c, pl.ds(s * _RPT, _RPT), :])

    kern = functools.partial(
        pl.kernel, mesh=_sc_mesh(),
        out_type=jax.ShapeDtypeStruct((2, NP, r_width), jnp.float32),
        compiler_params=_SC_PARAMS,
        scratch_types=[
            pltpu.VMEM((2, ch, r_width), jnp.float32),
            pltpu.VMEM((2, ch, 16), jnp.float32),
            pltpu.VMEM((2, ch), jnp.int32),
            pltpu.VMEM((2, ch), jnp.int32),
            pltpu.VMEM((2, ch), jnp.int32),
            pltpu.VMEM((2, ch), jnp.int32),
            pltpu.VMEM((1, 16), jnp.float32),
            pltpu.VMEM_SHARED((NP, r_width), jnp.float32),
            pltpu.SemaphoreType.DMA,
            pltpu.SemaphoreType.DMA,
            pltpu.SemaphoreType.DMA,
            pltpu.SemaphoreType.DMA,
            pltpu.SemaphoreType.DMA,
            pltpu.SemaphoreType.DMA,
        ],
    )
    return kern(body)


def kernel(x, edge_index0, edge_index1, W_in, b_in, fc_W0, bias0, attn_l0,
           attn_r0, res_W0, ln_g, ln_b, fc_W1, bias1, attn_l1, attn_r1):
    src0, dst0 = edge_index0[0], edge_index0[1]
    src1, dst1 = edge_index1[0], edge_index1[1]

    F0, res0, erp0, gm0 = _proj0(x, W_in, b_in, fc_W0, res_W0,
                                 attn_l0, attn_r0)
    S0 = _agg_kernel(_R0, True, 80)(src0, dst0, erp0, gm0,
                                    F0.reshape(2 * N, _R0))
    F1, erp1, gm1 = _mid(S0, res0, bias0.reshape(1, HEADS * HID),
                         ln_g.reshape(1, -1), ln_b.reshape(1, -1),
                         fc_W1, attn_l1, attn_r1)
    S1 = _agg_kernel(_R1, False, 128)(src1, dst1, erp1, gm1, F1)
    out = _fin(S1, bias1.reshape(1, OUT))
    return out


# speedup vs baseline: 82.9569x; 1.0038x over previous
import functools

import jax
import jax.numpy as jnp
from jax import lax
from jax.experimental import pallas as pl
from jax.experimental.pallas import tpu as pltpu
import jax.experimental.pallas.tpu_sc as plsc

N = 10000
NP = 10240
E = 320000
D_IN = 128
HID = 64
HEADS = 4
OUT = 64
NEG_SLOPE = 0.2

_BLK = 400
_R0 = 144
_R1 = 80
_NB = E // 128
_EPAD = 2560
_RPT = NP // 16


def _lrelu(x):
    return jnp.where(x > 0, x, NEG_SLOPE * x)


def _proj0_body(x_ref, Win_ref, bin_ref, fcW_ref, resW_ref, al_ref, ar_ref,
                F_ref, res_ref, erp_ref, gm_ref):
    i = pl.program_id(0)
    x = x_ref[...]
    h = jnp.dot(x, Win_ref[...], preferred_element_type=jnp.float32) + bin_ref[...]
    feat = jnp.dot(h, fcW_ref[...], preferred_element_type=jnp.float32)
    res_ref[...] = jnp.dot(h, resW_ref[...], preferred_element_type=jnp.float32)
    f = feat.reshape(_BLK, HEADS, HID)
    el = jnp.sum(f * al_ref[...][None], axis=-1)
    er = jnp.sum(f * ar_ref[...][None], axis=-1)
    zpad = jnp.zeros((_BLK, 16 - HEADS), jnp.float32)
    erp_ref[...] = jnp.concatenate([er, zpad], axis=1)
    @pl.when(i == 0)
    def _():
        gm_ref[...] = jnp.full((1, 16), -1e30, jnp.float32)
    gm_ref[...] = jnp.maximum(gm_ref[...], jnp.max(el))
    ones = jnp.ones((_BLK, 2), jnp.float32)
    fpad = jnp.zeros((_BLK, _R0 - 2 * HID - HEADS - 2), jnp.float32)
    F_ref[0] = jnp.concatenate([feat[:, :2 * HID], el, ones, fpad], axis=1)
    F_ref[1] = jnp.concatenate([feat[:, 2 * HID:], el, ones, fpad], axis=1)


def _proj0(x, W_in, b_in, fc_W0, res_W0, al0, ar0):
    return pl.pallas_call(
        _proj0_body,
        grid=(N // _BLK,),
        in_specs=[
            pl.BlockSpec((_BLK, D_IN), lambda i: (i, 0)),
            pl.BlockSpec((D_IN, HID), lambda i: (0, 0)),
            pl.BlockSpec((HID,), lambda i: (0,)),
            pl.BlockSpec((HID, HEADS * HID), lambda i: (0, 0)),
            pl.BlockSpec((HID, HEADS * HID), lambda i: (0, 0)),
            pl.BlockSpec((HEADS, HID), lambda i: (0, 0)),
            pl.BlockSpec((HEADS, HID), lambda i: (0, 0)),
        ],
        out_specs=[
            pl.BlockSpec((2, _BLK, _R0), lambda i: (0, i, 0)),
            pl.BlockSpec((_BLK, HEADS * HID), lambda i: (i, 0)),
            pl.BlockSpec((_BLK, 16), lambda i: (i, 0)),
            pl.BlockSpec((1, 16), lambda i: (0, 0)),
        ],
        out_shape=[
            jax.ShapeDtypeStruct((2, N, _R0), jnp.float32),
            jax.ShapeDtypeStruct((N, HEADS * HID), jnp.float32),
            jax.ShapeDtypeStruct((N, 16), jnp.float32),
            jax.ShapeDtypeStruct((1, 16), jnp.float32),
        ],
    )(x, W_in, b_in, fc_W0, res_W0, al0, ar0)


def _mid_body(Sp_ref, res_ref, bias_ref, lng_ref, lnb_ref,
              fcW1_ref, al1_ref, ar1_ref,
              F_ref, erp_ref, gm_ref):
    i = pl.program_id(0)
    Sa = Sp_ref[0]
    Sb = Sp_ref[1]
    dcol = 2 * HID + HEADS
    den = jnp.concatenate([Sa[:, dcol:dcol + 2],
                           Sb[:, dcol:dcol + 2]], axis=1)
    inv = 1.0 / jnp.maximum(den, 1e-9)
    inv = jnp.repeat(inv, HID, axis=1)
    S = jnp.concatenate([Sa[:, :2 * HID], Sb[:, :2 * HID]], axis=1)
    rst = S * inv + res_ref[...] + bias_ref[...]
    mu = jnp.mean(rst, axis=-1, keepdims=True)
    var = jnp.mean((rst - mu) ** 2, axis=-1, keepdims=True)
    hn = (rst - mu) / jnp.sqrt(var + 1e-5) * lng_ref[...] + lnb_ref[...]
    h = jnp.where(hn > 0, hn, jnp.exp(jnp.minimum(hn, 0.0)) - 1.0)
    feat = jnp.dot(h, fcW1_ref[...], preferred_element_type=jnp.float32)
    el = jnp.sum(feat * al1_ref[...], axis=-1, keepdims=True)
    er = jnp.sum(feat * ar1_ref[...], axis=-1, keepdims=True)
    zpad = jnp.zeros((_BLK, 15), jnp.float32)
    erp_ref[...] = jnp.concatenate([er, zpad], axis=1)
    @pl.when(i == 0)
    def _():
        gm_ref[...] = jnp.full((1, 16), -1e30, jnp.float32)
    gm_ref[...] = jnp.maximum(gm_ref[...], jnp.max(el))
    ones = jnp.ones((_BLK, 1), jnp.float32)
    fpad = jnp.zeros((_BLK, _R1 - OUT - 2), jnp.float32)
    F_ref[...] = jnp.concatenate([feat, el, ones, fpad], axis=1)


def _mid(S0p, res0, bias0, ln_g, ln_b, fc_W1, al1, ar1):
    return pl.pallas_call(
        _mid_body,
        grid=(N // _BLK,),
        in_specs=[
            pl.BlockSpec((2, _BLK, _R0), lambda i: (0, i, 0)),
            pl.BlockSpec((_BLK, HEADS * HID), lambda i: (i, 0)),
            pl.BlockSpec((1, HEADS * HID), lambda i: (0, 0)),
            pl.BlockSpec((1, HEADS * HID), lambda i: (0, 0)),
            pl.BlockSpec((1, HEADS * HID), lambda i: (0, 0)),
            pl.BlockSpec((HEADS * HID, OUT), lambda i: (0, 0)),
            pl.BlockSpec((1, OUT), lambda i: (0, 0)),
            pl.BlockSpec((1, OUT), lambda i: (0, 0)),
        ],
        out_specs=[
            pl.BlockSpec((_BLK, _R1), lambda i: (i, 0)),
            pl.BlockSpec((_BLK, 16), lambda i: (i, 0)),
            pl.BlockSpec((1, 16), lambda i: (0, 0)),
        ],
        out_shape=[
            jax.ShapeDtypeStruct((N, _R1), jnp.float32),
            jax.ShapeDtypeStruct((N, 16), jnp.float32),
            jax.ShapeDtypeStruct((1, 16), jnp.float32),
        ],
    )(S0p, res0, bias0, ln_g, ln_b, fc_W1, al1, ar1)


def _fin_body(Sp_ref, bias_ref, out_ref):
    agg = Sp_ref[0] + Sp_ref[1]
    den = jnp.maximum(agg[:, OUT + 1:OUT + 2], 1e-9)
    out_ref[...] = agg[:, :OUT] / den + bias_ref[...]


def _fin(S1p, bias1):
    return pl.pallas_call(
        _fin_body,
        grid=(N // _BLK,),
        in_specs=[
            pl.BlockSpec((2, _BLK, _R1), lambda i: (0, i, 0)),
            pl.BlockSpec((1, OUT), lambda i: (0, 0)),
        ],
        out_specs=pl.BlockSpec((_BLK, OUT), lambda i: (i, 0)),
        out_shape=jax.ShapeDtypeStruct((N, OUT), jnp.float32),
    )(S1p, bias1)


def _sc_mesh():
    return plsc.VectorSubcoreMesh(core_axis_name="c", subcore_axis_name="s")


_SC_PARAMS = pltpu.CompilerParams(use_tc_tiling_on_sc=False)


def _agg_kernel(r_width, heads_split, ch):
    nv = r_width // 16
    ng = ch // 16
    if heads_split:
        total_chunks = E // ch
    else:
        total_chunks = (E // 2) // ch
    nl = (total_chunks + 15) // 16
    nt2 = (nl + 1) // 2

    def body(src_hbm, dst_hbm, erp_hbm, gm_hbm, F_hbm, out_hbm,
             rows_v, er_v, src_v, dst_v, sdst_v, idx2, gm_v, S_sh,
             sl0, sl1, sg0, sg1, ss0, ss1):
        c = lax.axis_index("c")
        s = lax.axis_index("s")
        base = s * nl
        cnt = jnp.minimum(nl, total_chunks - s * nl)
        eoff0 = (0 if heads_split else c * (E // 2)) + base * ch
        sl = (sl0, sl1)
        sg = (sg0, sg1)
        ss = (ss0, ss1)

        def fire_lin(local, k):
            off = eoff0 + local * ch
            pltpu.async_copy(src_hbm.at[pl.ds(off, ch)], src_v.at[k], sl[k])
            pltpu.async_copy(dst_hbm.at[pl.ds(off, ch)], dst_v.at[k], sl[k])

        def drain_lin(k):
            pltpu.make_async_copy(src_hbm.at[pl.ds(0, ch)],
                                  src_v.at[k], sl[k]).wait()
            pltpu.make_async_copy(dst_hbm.at[pl.ds(0, ch)],
                                  dst_v.at[k], sl[k]).wait()

        def fire_gather(k):
            if heads_split:
                for g in range(ng):
                    sv = src_v[k, pl.ds(g * 16, 16)]
                    idx2[k, pl.ds(g * 16, 16)] = sv + c * N
                fidx = idx2.at[k]
            else:
                fidx = src_v.at[k]
            pltpu.async_copy(F_hbm.at[fidx], rows_v.at[k], sg[k])
            pltpu.async_copy(erp_hbm.at[dst_v.at[k]], er_v.at[k], sg[k])

        def drain_gather(k):
            pltpu.make_async_copy(F_hbm.at[pl.ds(0, ch), :],
                                  rows_v.at[k], sg[k]).wait()
            pltpu.make_async_copy(erp_hbm.at[pl.ds(0, ch), :],
                                  er_v.at[k], sg[k]).wait()

        def drain_scatter(k):
            pltpu.make_async_copy(rows_v.at[k], S_sh.at[sdst_v.at[k]],
                                  ss[k]).wait()

        def scale(k):
            @plsc.parallel_loop(0, ch, unroll=2)
            def _(j):
                el = rows_v[k, j, pl.ds((nv - 1) * 16, 16)]
                er = er_v[k, j, :]
                ee = jnp.exp(_lrelu(el + er) - _lrelu(gs + er))
                if heads_split:
                    m0s = jnp.where(c == 0, ee[0], ee[2])
                    m1s = jnp.where(c == 0, ee[1], ee[3])
                    dl0, dl1 = HEADS, HEADS + 1
                else:
                    m0s = ee[0]
                    m1s = ee[0]
                    dl0, dl1 = 1, 1
                m0 = jnp.full((16,), m0s, jnp.float32)
                m1 = jnp.full((16,), m1s, jnp.float32)
                half = (nv - 1) // 2 if heads_split else nv - 1
                for w in range(nv - 1):
                    r = rows_v[k, j, pl.ds(w * 16, 16)]
                    rows_v[k, j, pl.ds(w * 16, 16)] = r * (m0 if w < half
                                                           else m1)
                io = lax.broadcasted_iota(jnp.int32, (16,), 0)
                mult = jnp.where(io == dl0, m0, jnp.where(io == dl1, m1, 0.0))
                rows_v[k, j, pl.ds((nv - 1) * 16, 16)] = el * mult

        @pl.when(cnt > 0)
        def _():
            fire_lin(0, 0)

        @pl.when(cnt > 1)
        def _():
            fire_lin(1, 1)

        pltpu.sync_copy(gm_hbm, gm_v)
        gs = gm_v[0, :]

        @pl.when(cnt > 0)
        def _():
            drain_lin(0)
            fire_gather(0)

        def zrow(j, _):
            for v in range(nv):
                rows_v[1, j, pl.ds(v * 16, 16)] = jnp.zeros((16,), jnp.float32)
            return 0

        lax.fori_loop(0, ch, zrow, 0)
        for q in range(_RPT // ch):
            pltpu.sync_copy(rows_v.at[1],
                            S_sh.at[pl.ds(s * _RPT + q * ch, ch), :])
        plsc.subcore_barrier()

        def it(t, _):
            for k in (0, 1):
                local = t * 2 + k

                @pl.when((local >= 1) & (local < cnt))
                def _():
                    drain_scatter(1 - k)

                @pl.when(local + 1 < cnt)
                def _():
                    drain_lin(1 - k)
                    fire_gather(1 - k)

                @pl.when(local < cnt)
                def _():
                    drain_gather(k)
                    for g in range(ng):
                        sdst_v[k, pl.ds(g * 16, 16)] = \
                            dst_v[k, pl.ds(g * 16, 16)]

                @pl.when(local + 2 < cnt)
                def _():
                    fire_lin(local + 2, k)

                @pl.when(local < cnt)
                def _():
                    scale(k)
                    pltpu.async_copy(rows_v.at[k], S_sh.at[sdst_v.at[k]],
                                     ss[k], add=True)
            return 0

        lax.fori_loop(0, nt2, it, 0)
        for k in (0, 1):
            @pl.when((cnt >= 1) & (lax.rem(cnt - 1, 2) == k))
            def _():
                drain_scatter(k)
        plsc.subcore_barrier()
        pltpu.sync_copy(S_sh.at[pl.ds(s * _RPT, _RPT), :],
                        out_hbm.at[c, pl.ds(s * _RPT, _RPT), :])

    kern = functools.partial(
        pl.kernel, mesh=_sc_mesh(),
        out_type=jax.ShapeDtypeStruct((2, NP, r_width), jnp.float32),
        compiler_params=_SC_PARAMS,
        scratch_types=[
            pltpu.VMEM((2, ch, r_width), jnp.float32),
            pltpu.VMEM((2, ch, 16), jnp.float32),
            pltpu.VMEM((2, ch), jnp.int32),
            pltpu.VMEM((2, ch), jnp.int32),
            pltpu.VMEM((2, ch), jnp.int32),
            pltpu.VMEM((2, ch), jnp.int32),
            pltpu.VMEM((1, 16), jnp.float32),
            pltpu.VMEM_SHARED((NP, r_width), jnp.float32),
            pltpu.SemaphoreType.DMA,
            pltpu.SemaphoreType.DMA,
            pltpu.SemaphoreType.DMA,
            pltpu.SemaphoreType.DMA,
            pltpu.SemaphoreType.DMA,
            pltpu.SemaphoreType.DMA,
        ],
    )
    return kern(body)


def kernel(x, edge_index0, edge_index1, W_in, b_in, fc_W0, bias0, attn_l0,
           attn_r0, res_W0, ln_g, ln_b, fc_W1, bias1, attn_l1, attn_r1):
    src0, dst0 = edge_index0[0], edge_index0[1]
    src1, dst1 = edge_index1[0], edge_index1[1]

    F0, res0, erp0, gm0 = _proj0(x, W_in, b_in, fc_W0, res_W0,
                                 attn_l0, attn_r0)
    S0 = _agg_kernel(_R0, True, 80)(src0, dst0, erp0, gm0,
                                    F0.reshape(2 * N, _R0))
    F1, erp1, gm1 = _mid(S0, res0, bias0.reshape(1, HEADS * HID),
                         ln_g.reshape(1, -1), ln_b.reshape(1, -1),
                         fc_W1, attn_l1, attn_r1)
    S1 = _agg_kernel(_R1, False, 128)(src1, dst1, erp1, gm1, F1)
    out = _fin(S1, bias1.reshape(1, OUT))
    return out


# speedup vs baseline: 82.9889x; 1.0004x over previous
import functools

import jax
import jax.numpy as jnp
from jax import lax
from jax.experimental import pallas as pl
from jax.experimental.pallas import tpu as pltpu
import jax.experimental.pallas.tpu_sc as plsc

N = 10000
NP = 10240
E = 320000
D_IN = 128
HID = 64
HEADS = 4
OUT = 64
NEG_SLOPE = 0.2

_BLK = 400
_R0 = 144
_R1 = 80
_NB = E // 128
_RPT = NP // 16


def _lrelu(x):
    return jnp.where(x > 0, x, NEG_SLOPE * x)


def _proj0_body(x_ref, Win_ref, bin_ref, fcW_ref, resW_ref, al_ref, ar_ref,
                F_ref, res_ref, erp_ref, gm_ref):
    i = pl.program_id(0)
    x = x_ref[...]
    h = jnp.dot(x, Win_ref[...], preferred_element_type=jnp.float32) + bin_ref[...]
    feat = jnp.dot(h, fcW_ref[...], preferred_element_type=jnp.float32)
    res_ref[...] = jnp.dot(h, resW_ref[...], preferred_element_type=jnp.float32)
    f = feat.reshape(_BLK, HEADS, HID)
    el = jnp.sum(f * al_ref[...][None], axis=-1)
    er = jnp.sum(f * ar_ref[...][None], axis=-1)
    zpad = jnp.zeros((_BLK, 16 - HEADS), jnp.float32)
    erp_ref[...] = jnp.concatenate([er, zpad], axis=1)
    @pl.when(i == 0)
    def _():
        gm_ref[...] = jnp.full((1, 16), -1e30, jnp.float32)
    gm_ref[...] = jnp.maximum(gm_ref[...], jnp.max(el))
    ones = jnp.ones((_BLK, 2), jnp.float32)
    fpad = jnp.zeros((_BLK, _R0 - 2 * HID - HEADS - 2), jnp.float32)
    F_ref[0] = jnp.concatenate([feat[:, :2 * HID], el, ones, fpad], axis=1)
    F_ref[1] = jnp.concatenate([feat[:, 2 * HID:], el, ones, fpad], axis=1)


def _proj0(x, W_in, b_in, fc_W0, res_W0, al0, ar0):
    return pl.pallas_call(
        _proj0_body,
        grid=(N // _BLK,),
        in_specs=[
            pl.BlockSpec((_BLK, D_IN), lambda i: (i, 0)),
            pl.BlockSpec((D_IN, HID), lambda i: (0, 0)),
            pl.BlockSpec((HID,), lambda i: (0,)),
            pl.BlockSpec((HID, HEADS * HID), lambda i: (0, 0)),
            pl.BlockSpec((HID, HEADS * HID), lambda i: (0, 0)),
            pl.BlockSpec((HEADS, HID), lambda i: (0, 0)),
            pl.BlockSpec((HEADS, HID), lambda i: (0, 0)),
        ],
        out_specs=[
            pl.BlockSpec((2, _BLK, _R0), lambda i: (0, i, 0)),
            pl.BlockSpec((_BLK, HEADS * HID), lambda i: (i, 0)),
            pl.BlockSpec((_BLK, 16), lambda i: (i, 0)),
            pl.BlockSpec((1, 16), lambda i: (0, 0)),
        ],
        out_shape=[
            jax.ShapeDtypeStruct((2, N, _R0), jnp.float32),
            jax.ShapeDtypeStruct((N, HEADS * HID), jnp.float32),
            jax.ShapeDtypeStruct((N, 16), jnp.float32),
            jax.ShapeDtypeStruct((1, 16), jnp.float32),
        ],
    )(x, W_in, b_in, fc_W0, res_W0, al0, ar0)


def _mid_body(Sp_ref, res_ref, bias_ref, lng_ref, lnb_ref,
              fcW1_ref, al1_ref, ar1_ref,
              F_ref, erp_ref, gm_ref):
    i = pl.program_id(0)
    Sa = Sp_ref[0]
    Sb = Sp_ref[1]
    dcol = 2 * HID + HEADS
    den = jnp.concatenate([Sa[:, dcol:dcol + 2],
                           Sb[:, dcol:dcol + 2]], axis=1)
    inv = 1.0 / jnp.maximum(den, 1e-9)
    inv = jnp.repeat(inv, HID, axis=1)
    S = jnp.concatenate([Sa[:, :2 * HID], Sb[:, :2 * HID]], axis=1)
    rst = S * inv + res_ref[...] + bias_ref[...]
    mu = jnp.mean(rst, axis=-1, keepdims=True)
    var = jnp.mean((rst - mu) ** 2, axis=-1, keepdims=True)
    hn = (rst - mu) / jnp.sqrt(var + 1e-5) * lng_ref[...] + lnb_ref[...]
    h = jnp.where(hn > 0, hn, jnp.exp(jnp.minimum(hn, 0.0)) - 1.0)
    feat = jnp.dot(h, fcW1_ref[...], preferred_element_type=jnp.float32)
    el = jnp.sum(feat * al1_ref[...], axis=-1, keepdims=True)
    er = jnp.sum(feat * ar1_ref[...], axis=-1, keepdims=True)
    zpad = jnp.zeros((_BLK, 15), jnp.float32)
    erp_ref[...] = jnp.concatenate([er, zpad], axis=1)
    @pl.when(i == 0)
    def _():
        gm_ref[...] = jnp.full((1, 16), -1e30, jnp.float32)
    gm_ref[...] = jnp.maximum(gm_ref[...], jnp.max(el))
    ones = jnp.ones((_BLK, 1), jnp.float32)
    fpad = jnp.zeros((_BLK, _R1 - OUT - 2), jnp.float32)
    F_ref[...] = jnp.concatenate([feat, el, ones, fpad], axis=1)


def _mid(S0p, res0, bias0, ln_g, ln_b, fc_W1, al1, ar1):
    return pl.pallas_call(
        _mid_body,
        grid=(N // _BLK,),
        in_specs=[
            pl.BlockSpec((2, _BLK, _R0), lambda i: (0, i, 0)),
            pl.BlockSpec((_BLK, HEADS * HID), lambda i: (i, 0)),
            pl.BlockSpec((1, HEADS * HID), lambda i: (0, 0)),
            pl.BlockSpec((1, HEADS * HID), lambda i: (0, 0)),
            pl.BlockSpec((1, HEADS * HID), lambda i: (0, 0)),
            pl.BlockSpec((HEADS * HID, OUT), lambda i: (0, 0)),
            pl.BlockSpec((1, OUT), lambda i: (0, 0)),
            pl.BlockSpec((1, OUT), lambda i: (0, 0)),
        ],
        out_specs=[
            pl.BlockSpec((_BLK, _R1), lambda i: (i, 0)),
            pl.BlockSpec((_BLK, 16), lambda i: (i, 0)),
            pl.BlockSpec((1, 16), lambda i: (0, 0)),
        ],
        out_shape=[
            jax.ShapeDtypeStruct((N, _R1), jnp.float32),
            jax.ShapeDtypeStruct((N, 16), jnp.float32),
            jax.ShapeDtypeStruct((1, 16), jnp.float32),
        ],
    )(S0p, res0, bias0, ln_g, ln_b, fc_W1, al1, ar1)


def _fin_body(Sp_ref, bias_ref, out_ref):
    agg = Sp_ref[0] + Sp_ref[1]
    den = jnp.maximum(agg[:, OUT + 1:OUT + 2], 1e-9)
    out_ref[...] = agg[:, :OUT] / den + bias_ref[...]


def _fin(S1p, bias1):
    return pl.pallas_call(
        _fin_body,
        grid=(N // _BLK,),
        in_specs=[
            pl.BlockSpec((2, _BLK, _R1), lambda i: (0, i, 0)),
            pl.BlockSpec((1, OUT), lambda i: (0, 0)),
        ],
        out_specs=pl.BlockSpec((_BLK, OUT), lambda i: (i, 0)),
        out_shape=jax.ShapeDtypeStruct((N, OUT), jnp.float32),
    )(S1p, bias1)


def _sc_mesh():
    return plsc.VectorSubcoreMesh(core_axis_name="c", subcore_axis_name="s")


_SC_PARAMS = pltpu.CompilerParams(use_tc_tiling_on_sc=False)


def _agg_kernel(r_width, heads_split, ch):
    nv = r_width // 16
    ng = ch // 16
    if heads_split:
        total_chunks = E // ch
    else:
        total_chunks = (E // 2) // ch
    nl = (total_chunks + 15) // 16
    nt2 = (nl + 1) // 2

    def body(src_hbm, dst_hbm, erp_hbm, gm_hbm, F_hbm, out_hbm,
             rows_v, er_v, src_v, dst_v, sdst_v, idx2, gm_v, S_sh,
             sl0, sl1, sg0, sg1, ss0, ss1):
        c = lax.axis_index("c")
        s = lax.axis_index("s")
        base = s * nl
        cnt = jnp.minimum(nl, total_chunks - s * nl)
        eoff0 = (0 if heads_split else c * (E // 2)) + base * ch
        sl = (sl0, sl1)
        sg = (sg0, sg1)
        ss = (ss0, ss1)

        def fire_lin(local, k):
            off = eoff0 + local * ch
            pltpu.async_copy(src_hbm.at[pl.ds(off, ch)], src_v.at[k], sl[k])
            pltpu.async_copy(dst_hbm.at[pl.ds(off, ch)], dst_v.at[k], sl[k])

        def drain_lin(k):
            pltpu.make_async_copy(src_hbm.at[pl.ds(0, ch)],
                                  src_v.at[k], sl[k]).wait()
            pltpu.make_async_copy(dst_hbm.at[pl.ds(0, ch)],
                                  dst_v.at[k], sl[k]).wait()

        def fire_gather(k):
            if heads_split:
                for g in range(ng):
                    sv = src_v[k, pl.ds(g * 16, 16)]
                    idx2[k, pl.ds(g * 16, 16)] = sv + c * N
                fidx = idx2.at[k]
            else:
                fidx = src_v.at[k]
            pltpu.async_copy(F_hbm.at[fidx], rows_v.at[k], sg[k])
            pltpu.async_copy(erp_hbm.at[dst_v.at[k]], er_v.at[k], sg[k])

        def drain_gather(k):
            pltpu.make_async_copy(F_hbm.at[pl.ds(0, ch), :],
                                  rows_v.at[k], sg[k]).wait()
            pltpu.make_async_copy(erp_hbm.at[pl.ds(0, ch), :],
                                  er_v.at[k], sg[k]).wait()

        def drain_scatter(k):
            pltpu.make_async_copy(rows_v.at[k], S_sh.at[sdst_v.at[k]],
                                  ss[k]).wait()

        def scale(k):
            @plsc.parallel_loop(0, ch, unroll=2)
            def _(j):
                el = rows_v[k, j, pl.ds((nv - 1) * 16, 16)]
                er = er_v[k, j, :]
                ee = jnp.exp(_lrelu(el + er) - _lrelu(gs + er))
                if heads_split:
                    m0s = jnp.where(c == 0, ee[0], ee[2])
                    m1s = jnp.where(c == 0, ee[1], ee[3])
                    dl0, dl1 = HEADS, HEADS + 1
                else:
                    m0s = ee[0]
                    m1s = ee[0]
                    dl0, dl1 = 1, 1
                m0 = jnp.full((16,), m0s, jnp.float32)
                m1 = jnp.full((16,), m1s, jnp.float32)
                half = (nv - 1) // 2 if heads_split else nv - 1
                for w in range(nv - 1):
                    r = rows_v[k, j, pl.ds(w * 16, 16)]
                    rows_v[k, j, pl.ds(w * 16, 16)] = r * (m0 if w < half
                                                           else m1)
                io = lax.broadcasted_iota(jnp.int32, (16,), 0)
                mult = jnp.where(io == dl0, m0, jnp.where(io == dl1, m1, 0.0))
                rows_v[k, j, pl.ds((nv - 1) * 16, 16)] = el * mult

        @pl.when(cnt > 0)
        def _():
            fire_lin(0, 0)

        @pl.when(cnt > 1)
        def _():
            fire_lin(1, 1)

        pltpu.sync_copy(gm_hbm, gm_v)
        gs = gm_v[0, :]

        @pl.when(cnt > 0)
        def _():
            drain_lin(0)
            fire_gather(0)

        def zrow(j, _):
            for v in range(nv):
                rows_v[1, j, pl.ds(v * 16, 16)] = jnp.zeros((16,), jnp.float32)
            return 0

        lax.fori_loop(0, ch, zrow, 0)
        for q in range(_RPT // ch):
            pltpu.sync_copy(rows_v.at[1],
                            S_sh.at[pl.ds(s * _RPT + q * ch, ch), :])
        plsc.subcore_barrier()

        def it(t, _):
            for k in (0, 1):
                local = t * 2 + k

                @pl.when((local >= 1) & (local < cnt))
                def _():
                    drain_scatter(1 - k)

                @pl.when(local + 1 < cnt)
                def _():
                    drain_lin(1 - k)
                    fire_gather(1 - k)

                @pl.when(local < cnt)
                def _():
                    drain_gather(k)
                    for g in range(ng):
                        sdst_v[k, pl.ds(g * 16, 16)] = \
                            dst_v[k, pl.ds(g * 16, 16)]

                @pl.when(local + 2 < cnt)
                def _():
                    fire_lin(local + 2, k)

                @pl.when(local < cnt)
                def _():
                    scale(k)
                    pltpu.async_copy(rows_v.at[k], S_sh.at[sdst_v.at[k]],
                                     ss[k], add=True)
            return 0

        lax.fori_loop(0, nt2, it, 0)
        for k in (0, 1):
            @pl.when((cnt >= 1) & (lax.rem(cnt - 1, 2) == k))
            def _():
                drain_scatter(k)
        plsc.subcore_barrier()
        pltpu.sync_copy(S_sh.at[pl.ds(s * _RPT, _RPT), :],
                        out_hbm.at[c, pl.ds(s * _RPT, _RPT), :])

    kern = functools.partial(
        pl.kernel, mesh=_sc_mesh(),
        out_type=jax.ShapeDtypeStruct((2, NP, r_width), jnp.float32),
        compiler_params=_SC_PARAMS,
        scratch_types=[
            pltpu.VMEM((2, ch, r_width), jnp.float32),
            pltpu.VMEM((2, ch, 16), jnp.float32),
            pltpu.VMEM((2, ch), jnp.int32),
            pltpu.VMEM((2, ch), jnp.int32),
            pltpu.VMEM((2, ch), jnp.int32),
            pltpu.VMEM((2, ch), jnp.int32),
            pltpu.VMEM((1, 16), jnp.float32),
            pltpu.VMEM_SHARED((NP, r_width), jnp.float32),
            pltpu.SemaphoreType.DMA,
            pltpu.SemaphoreType.DMA,
            pltpu.SemaphoreType.DMA,
            pltpu.SemaphoreType.DMA,
            pltpu.SemaphoreType.DMA,
            pltpu.SemaphoreType.DMA,
        ],
    )
    return kern(body)


def kernel(x, edge_index0, edge_index1, W_in, b_in, fc_W0, bias0, attn_l0,
           attn_r0, res_W0, ln_g, ln_b, fc_W1, bias1, attn_l1, attn_r1):
    src0, dst0 = edge_index0[0], edge_index0[1]
    src1, dst1 = edge_index1[0], edge_index1[1]

    F0, res0, erp0, gm0 = _proj0(x, W_in, b_in, fc_W0, res_W0,
                                 attn_l0, attn_r0)
    S0 = _agg_kernel(_R0, True, 80)(src0, dst0, erp0, gm0,
                                    F0.reshape(2 * N, _R0))
    F1, erp1, gm1 = _mid(S0, res0, bias0.reshape(1, HEADS * HID),
                         ln_g.reshape(1, -1), ln_b.reshape(1, -1),
                         fc_W1, attn_l1, attn_r1)
    S1 = _agg_kernel(_R1, False, 128)(src1, dst1, erp1, gm1, F1)
    out = _fin(S1, bias1.reshape(1, OUT))
    return out
